# baseline (speedup 1.0000x reference)
import functools

import jax
import jax.numpy as jnp
from jax import lax
from jax.experimental import pallas as pl
from jax.experimental.pallas import tpu as pltpu
from jax.experimental.pallas import tpu_sc as plsc

N = 10000
E = 320000
EP = E + N
IN_DIMS = [3, 32, 64, 128, 256]
N_CLUSTERS = 30

NC, NS = 2, 16
NW = NC * NS
EPT = 10112
EPAD = EPT * NW
NPAD = 10240
ROWS_PT = NPAD // NS

_mesh = plsc.VectorSubcoreMesh(
    core_axis_name="c", subcore_axis_name="s", num_cores=NC, num_subcores=NS)
_sc_params = pltpu.CompilerParams(use_tc_tiling_on_sc=False)


def _geom(cc, kk):
    nch_arr = EPT // cc

    def bound(real):
        gch = -(-real // cc)
        b = -(-gch // NW)
        return min(-(-b // kk) * kk, nch_arr)

    return nch_arr, bound(E)


def _stage_idx(src_h, dst_h, core, sub, idxs, idxd):
    pltpu.sync_copy(src_h.at[core, sub], idxs)
    pltpu.sync_copy(dst_h.at[core, sub], idxd)


def _zero_acc(zc, acc, sub):
    pltpu.sync_copy(zc, acc.at[pl.ds(sub * ROWS_PT, ROWS_PT)])


def _copy_out(acc, out_ref, sub):
    pltpu.sync_copy(acc.at[pl.ds(sub * ROWS_PT, ROWS_PT)],
                    out_ref.at[pl.ds(sub * ROWS_PT, ROWS_PT)])


def _pipe_phase(tbl, idxs, idxd, bufs, sems, acc, nch, kk):
    @pl.loop(0, nch, step=kk)
    def _(j):
        descs = [
            pltpu.async_copy(tbl.at[idxs.at[j + b]], bufs[b], sems[b])
            for b in range(kk)
        ]
        for b in range(kk):
            descs[b].wait()
            pltpu.sync_copy(bufs[b], acc.at[idxd.at[j + b]], add=True)


def _make_l1(cc=128, kk=1):
    d = 16
    nch_arr, nch_e = _geom(cc, kk)

    @functools.partial(
        pl.kernel,
        out_type=jax.ShapeDtypeStruct((NC, 2, NPAD, d), jnp.float32),
        mesh=_mesh,
        compiler_params=_sc_params,
        scratch_types=[
            pltpu.VMEM((nch_arr, cc), jnp.int32),
            pltpu.VMEM((nch_arr, cc), jnp.int32),
            [pltpu.VMEM((cc, d), jnp.float32)] * kk,
            pltpu.VMEM((cc, d), jnp.float32),
            pltpu.VMEM_SHARED((NPAD, d), jnp.float32),
            [pltpu.SemaphoreType.DMA] * kk,
        ],
    )
    def k(t1, src_h, dst_h, zc, ones_h, out, idxs, idxd, bufs, ones_v, acc,
          sems):
        core = lax.axis_index("c")
        sub = lax.axis_index("s")
        nch = nch_e
        _stage_idx(src_h, dst_h, core, sub, idxs, idxd)
        pltpu.sync_copy(ones_h, ones_v)

        _zero_acc(zc, acc, sub)
        plsc.subcore_barrier()
        _pipe_phase(t1, idxs, idxd, bufs, sems, acc, nch, kk)
        plsc.subcore_barrier()
        _copy_out(acc, out.at[core, 0], sub)

        _zero_acc(zc, acc, sub)
        plsc.subcore_barrier()

        @pl.loop(0, nch)
        def _(j):
            pltpu.sync_copy(ones_v, acc.at[idxd.at[j]], add=True)

        plsc.subcore_barrier()
        _copy_out(acc, out.at[core, 1], sub)
        plsc.subcore_barrier()

    return k


def _make_mid(w, cc, kk):
    nch_arr, nch_e = _geom(cc, kk)

    @functools.partial(
        pl.kernel,
        out_type=jax.ShapeDtypeStruct((NC, NPAD, w), jnp.float32),
        mesh=_mesh,
        compiler_params=_sc_params,
        scratch_types=[
            pltpu.VMEM((nch_arr, cc), jnp.int32),
            pltpu.VMEM((nch_arr, cc), jnp.int32),
            [pltpu.VMEM((cc, w), jnp.float32)] * kk,
            pltpu.VMEM_SHARED((NPAD, w), jnp.float32),
            [pltpu.SemaphoreType.DMA] * kk,
        ],
    )
    def k(tbl, src_h, dst_h, zc, out, idxs, idxd, bufs, acc, sems):
        core = lax.axis_index("c")
        sub = lax.axis_index("s")
        nch = nch_e
        _stage_idx(src_h, dst_h, core, sub, idxs, idxd)
        _zero_acc(zc, acc, sub)
        plsc.subcore_barrier()
        _pipe_phase(tbl, idxs, idxd, bufs, sems, acc, nch, kk)
        plsc.subcore_barrier()
        _copy_out(acc, out.at[core], sub)
        plsc.subcore_barrier()

    return k


def _make_l4(cc=128, kk=1):
    d = 128
    nch_arr, nch_e = _geom(cc, kk)

    @functools.partial(
        pl.kernel,
        out_type=jax.ShapeDtypeStruct((NC, 2, NPAD, d), jnp.float32),
        mesh=_mesh,
        compiler_params=_sc_params,
        scratch_types=[
            pltpu.VMEM((nch_arr, cc), jnp.int32),
            pltpu.VMEM((nch_arr, cc), jnp.int32),
            [pltpu.VMEM((cc, d), jnp.float32)] * kk,
            pltpu.VMEM_SHARED((NPAD, d), jnp.float32),
            [pltpu.SemaphoreType.DMA] * kk,
        ],
    )
    def k(t1, t2, src_h, dst_h, zc, out, idxs, idxd, bufs, acc, sems):
        core = lax.axis_index("c")
        sub = lax.axis_index("s")
        nch = nch_e
        _stage_idx(src_h, dst_h, core, sub, idxs, idxd)
        for p, tp in ((0, t1), (1, t2)):
            _zero_acc(zc, acc, sub)
            plsc.subcore_barrier()
            _pipe_phase(tp, idxs, idxd, bufs, sems, acc, nch, kk)
            plsc.subcore_barrier()
            _copy_out(acc, out.at[core, p], sub)
        plsc.subcore_barrier()

    return k


_TCB = 2000


def _dot(a, b):
    return jnp.dot(a, b, preferred_element_type=jnp.float32)


def _l1_body(so, co, x, wp, bp, wn, bn, h, o):
    rcp = 1.0 / (co[...][0, :, 0:1] + 1.0)
    rcn = 1.0 / jnp.maximum(co[...][1, :, 0:1], 1.0)
    xv = x[...]
    agg_p = (so[...][0][:, :3] + xv) * rcp
    agg_n = so[...][1][:, :3] * rcn
    wpv = wp[...]
    wnv = wn[...]
    out_p = _dot(agg_p, wpv[:3]) + _dot(xv, wpv[3:]) + bp[...]
    out_n = _dot(agg_n, wnv[:3]) + _dot(xv, wnv[3:]) + bn[...]
    hv = h[...]
    o[...] = jnp.concatenate(
        [(jnp.maximum(out_p, 0.0) + hv) * 0.5,
         (jnp.maximum(out_n, 0.0) + hv) * 0.5], axis=1)


def _mid_body(d, split, sp, sn, co, z, wp, bp, wn, bn, h, *outs):
    rcp = 1.0 / (co[...][0, :, 0:1] + 1.0)
    rcn = 1.0 / jnp.maximum(co[...][1, :, 0:1], 1.0)
    spv = sp[...]
    snv = sn[...]
    zv = z[...]
    ap1 = (spv[:, :d] + zv[:, :d]) * rcp
    ap2 = (spv[:, d:] + zv[:, d:]) * rcp
    an1 = snv[:, :d] * rcn
    an2 = snv[:, d:] * rcn
    wpv = wp[...]
    wnv = wn[...]
    out_p = (_dot(ap1, wpv[:d]) + _dot(an2, wpv[d:2 * d])
             + _dot(zv[:, :d], wpv[2 * d:]) + bp[...])
    out_n = (_dot(ap2, wnv[:d]) + _dot(an1, wnv[d:2 * d])
             + _dot(zv[:, d:], wnv[2 * d:]) + bn[...])
    hv = h[...]
    zp = (jnp.maximum(out_p, 0.0) + hv) * 0.5
    zn = (jnp.maximum(out_n, 0.0) + hv) * 0.5
    if split:
        outs[0][...] = zp
        outs[1][...] = zn
    else:
        outs[0][...] = jnp.concatenate([zp, zn], axis=1)


def _l4_body(d, s4, co, z1, z2, wp, bp, wn, bn, h, o1, o2):
    rcp = 1.0 / (co[...][0, :, 0:1] + 1.0)
    rcn = 1.0 / jnp.maximum(co[...][1, :, 0:1], 1.0)
    s4v = s4[...]
    ap1 = (s4v[0, 0] + z1[...]) * rcp
    ap2 = (s4v[0, 1] + z2[...]) * rcp
    an1 = s4v[1, 0] * rcn
    an2 = s4v[1, 1] * rcn
    wpv = wp[...]
    wnv = wn[...]
    out_p = (_dot(ap1, wpv[:d]) + _dot(an2, wpv[d:2 * d])
             + _dot(z1[...], wpv[2 * d:]) + bp[...])
    out_n = (_dot(ap2, wnv[:d]) + _dot(an1, wnv[d:2 * d])
             + _dot(z2[...], wnv[2 * d:]) + bn[...])
    hv = h[...]
    o1[...] = (jnp.maximum(out_p, 0.0) + hv) * 0.5
    o2[...] = (jnp.maximum(out_n, 0.0) + hv) * 0.5


def _decq_body(z1, z2, mu, q1, q2):
    muv = mu[...]
    mu2 = jnp.sum(muv * muv, axis=1)[None, :]
    for z, q in ((z1, q1), (z2, q2)):
        zv = z[...]
        z2s = jnp.sum(zv * zv, axis=1, keepdims=True)
        cross = lax.dot_general(zv, muv, (((1,), (1,)), ((), ())),
                                preferred_element_type=jnp.float32)
        d2 = z2s + mu2 - 2.0 * cross
        qv = 1.0 / (1.0 + jnp.maximum(d2, 0.0))
        q[...] = qv / jnp.sum(qv, axis=1, keepdims=True)


def _row_spec(cols):
    return pl.BlockSpec((_TCB, cols), lambda i: (i, 0))


def _full_spec(r, c):
    return pl.BlockSpec((r, c), lambda i: (0, 0))


_cnt_spec = pl.BlockSpec((2, _TCB, 16), lambda i: (0, i, 0), )


def _tc_layer1(s1, x, wp, bp, wn, bn, h):
    g = N // _TCB
    dout = 32
    return pl.pallas_call(
        _l1_body,
        grid=(g,),
        in_specs=[
            pl.BlockSpec((2, None, _TCB, 16), lambda i: (0, 0, i, 0)),
            pl.BlockSpec((2, None, _TCB, 16), lambda i: (0, 1, i, 0)),
            _row_spec(3),
            _full_spec(6, dout), _full_spec(1, dout),
            _full_spec(6, dout), _full_spec(1, dout),
            _row_spec(dout),
        ],
        out_specs=pl.BlockSpec((_TCB, 2 * dout), lambda i: (i, 0)),
        out_shape=jax.ShapeDtypeStruct((N, 2 * dout), jnp.float32),
    )(s1, s1, x, wp, bp, wn, bn, h)


def _tc_mid(d, dout, split, s, s1, z, wp, bp, wn, bn, h):
    g = N // _TCB
    if split:
        out_specs = [_row_spec(dout), _row_spec(dout)]
        out_shape = [jax.ShapeDtypeStruct((N, dout), jnp.float32)] * 2
    else:
        out_specs = pl.BlockSpec((_TCB, 2 * dout), lambda i: (i, 0))
        out_shape = jax.ShapeDtypeStruct((N, 2 * dout), jnp.float32)
    return pl.pallas_call(
        functools.partial(_mid_body, d, split),
        grid=(g,),
        in_specs=[
            pl.BlockSpec((None, _TCB, 2 * d), lambda i: (0, i, 0)),
            pl.BlockSpec((None, _TCB, 2 * d), lambda i: (1, i, 0)),
            pl.BlockSpec((2, None, _TCB, 16), lambda i: (0, 1, i, 0)),
            _row_spec(2 * d),
            _full_spec(3 * d, dout), _full_spec(1, dout),
            _full_spec(3 * d, dout), _full_spec(1, dout),
            _row_spec(dout),
        ],
        out_specs=out_specs,
        out_shape=out_shape,
    )(s, s, s1, z, wp, bp, wn, bn, h)


def _tc_l4(s4, s1, z1, z2, wp, bp, wn, bn, h):
    g = N // _TCB
    d, dout = 128, 256
    return pl.pallas_call(
        functools.partial(_l4_body, d),
        grid=(g,),
        in_specs=[
            pl.BlockSpec((2, 2, _TCB, d), lambda i: (0, 0, i, 0)),
            pl.BlockSpec((2, None, _TCB, 16), lambda i: (0, 1, i, 0)),
            _row_spec(d), _row_spec(d),
            _full_spec(3 * d, dout), _full_spec(1, dout),
            _full_spec(3 * d, dout), _full_spec(1, dout),
            _row_spec(dout),
        ],
        out_specs=[_row_spec(dout), _row_spec(dout)],
        out_shape=[jax.ShapeDtypeStruct((N, dout), jnp.float32)] * 2,
    )(s4, s1, z1, z2, wp, bp, wn, bn, h)


def _tc_decq(z1, z2, mu):
    g = N // _TCB
    d = IN_DIMS[4]
    return pl.pallas_call(
        _decq_body,
        grid=(g,),
        in_specs=[
            _row_spec(d), _row_spec(d),
            _full_spec(N_CLUSTERS, d),
        ],
        out_specs=[_row_spec(N_CLUSTERS), _row_spec(N_CLUSTERS)],
        out_shape=[jax.ShapeDtypeStruct((N, N_CLUSTERS), jnp.float32)] * 2,
    )(z1, z2, mu)


def _pad_edges(src, dst):
    pad = EPAD - src.shape[0]
    src_p = jnp.concatenate([src, jnp.zeros((pad,), jnp.int32)])
    dst_p = jnp.concatenate([dst, jnp.full((pad,), N, jnp.int32)])
    return src_p, dst_p


def _tile_layout(arr, cc):
    return arr.reshape(EPT // cc, NW, cc).transpose(1, 0, 2)


def kernel(x, edge_index_sim, edge_index_disim, h1, h2, h3, h4,
           W1p, b1p, W1n, b1n, W2p, b2p, W2n, b2n, W3p, b3p, W3n, b3n,
           W4p, b4p, W4n, b4n, mu):
    sp_s, dp_s = _pad_edges(edge_index_sim[0], edge_index_sim[1])
    sn_s, dn_s = _pad_edges(edge_index_disim[0], edge_index_disim[1])
    src128 = jnp.stack([_tile_layout(sp_s, 128), _tile_layout(sn_s, 128)])
    dst128 = jnp.stack([_tile_layout(dp_s, 128), _tile_layout(dn_s, 128)])

    x16 = jnp.pad(x, ((0, 0), (0, 13)))
    ones16 = jnp.ones((128, 16), jnp.float32)

    def zc(w):
        return jnp.zeros((ROWS_PT, w), jnp.float32)

    s1 = _make_l1()(x16, src128, dst128, zc(16), ones16)
    z1 = _tc_layer1(s1, x, W1p, b1p.reshape(1, -1), W1n, b1n.reshape(1, -1),
                    h1)

    s2 = _make_mid(64, 128, 1)(z1, src128, dst128, zc(64))
    z2 = _tc_mid(32, 64, False, s2, s1, z1,
                 W2p, b2p.reshape(1, -1), W2n, b2n.reshape(1, -1), h2)

    s3 = _make_mid(128, 128, 1)(z2, src128, dst128, zc(128))
    z3a, z3b = _tc_mid(64, 128, True, s3, s1, z2,
                       W3p, b3p.reshape(1, -1), W3n, b3n.reshape(1, -1), h3)

    s4 = _make_l4()(z3a, z3b, src128, dst128, zc(128))
    z4a, z4b = _tc_l4(s4, s1, z3a, z3b,
                      W4p, b4p.reshape(1, -1), W4n, b4n.reshape(1, -1), h4)

    return _tc_decq(z4a, z4b, mu)


# speedup vs baseline: 1.4129x; 1.4129x over previous
import functools

import jax
import jax.numpy as jnp
from jax import lax
from jax.experimental import pallas as pl
from jax.experimental.pallas import tpu as pltpu
from jax.experimental.pallas import tpu_sc as plsc

N = 10000
E = 320000
EP = E + N
IN_DIMS = [3, 32, 64, 128, 256]
N_CLUSTERS = 30

NC, NS = 2, 16
NW = NC * NS
EPT = 10112
EPAD = EPT * NW
NPAD = 10240
ROWS_PT = NPAD // NS

_mesh = plsc.VectorSubcoreMesh(
    core_axis_name="c", subcore_axis_name="s", num_cores=NC, num_subcores=NS)
_sc_params = pltpu.CompilerParams(use_tc_tiling_on_sc=False)


def _geom(cc, kk):
    nch_arr = EPT // cc

    def bound(real):
        gch = -(-real // cc)
        b = -(-gch // NW)
        return min(-(-b // kk) * kk, nch_arr)

    return nch_arr, bound(E)


def _stage_idx(src_h, dst_h, core, sub, idxs, idxd):
    pltpu.sync_copy(src_h.at[core, sub], idxs)
    pltpu.sync_copy(dst_h.at[core, sub], idxd)


def _zero_acc(zc, acc, sub):
    pltpu.sync_copy(zc, acc.at[pl.ds(sub * ROWS_PT, ROWS_PT)])


def _copy_out(acc, out_ref, sub):
    pltpu.sync_copy(acc.at[pl.ds(sub * ROWS_PT, ROWS_PT)],
                    out_ref.at[pl.ds(sub * ROWS_PT, ROWS_PT)])


def _pipe_phase(tbl, idxs, idxd, bufs, sems, acc, nch, kk):
    @pl.loop(0, nch, step=kk)
    def _(j):
        descs = [
            pltpu.async_copy(tbl.at[idxs.at[j + b]], bufs[b], sems[b])
            for b in range(kk)
        ]
        for b in range(kk):
            descs[b].wait()
            pltpu.sync_copy(bufs[b], acc.at[idxd.at[j + b]], add=True)


def _make_l1(cc=128, kk=1):
    d = 16
    nch_arr, nch_e = _geom(cc, kk)

    @functools.partial(
        pl.kernel,
        out_type=jax.ShapeDtypeStruct((NC, 2, NPAD, d), jnp.float32),
        mesh=_mesh,
        compiler_params=_sc_params,
        scratch_types=[
            pltpu.VMEM((nch_arr, cc), jnp.int32),
            pltpu.VMEM((nch_arr, cc), jnp.int32),
            [pltpu.VMEM((cc, d), jnp.float32)] * kk,
            pltpu.VMEM((cc, d), jnp.float32),
            pltpu.VMEM_SHARED((NPAD, d), jnp.float32),
            [pltpu.SemaphoreType.DMA] * kk,
        ],
    )
    def k(t1, src_h, dst_h, zc, ones_h, out, idxs, idxd, bufs, ones_v, acc,
          sems):
        core = lax.axis_index("c")
        sub = lax.axis_index("s")
        nch = nch_e
        _stage_idx(src_h, dst_h, core, sub, idxs, idxd)
        pltpu.sync_copy(ones_h, ones_v)

        _zero_acc(zc, acc, sub)
        plsc.subcore_barrier()
        _pipe_phase(t1, idxs, idxd, bufs, sems, acc, nch, kk)
        plsc.subcore_barrier()
        _copy_out(acc, out.at[core, 0], sub)

        _zero_acc(zc, acc, sub)
        plsc.subcore_barrier()

        @pl.loop(0, nch)
        def _(j):
            pltpu.sync_copy(ones_v, acc.at[idxd.at[j]], add=True)

        plsc.subcore_barrier()
        _copy_out(acc, out.at[core, 1], sub)
        plsc.subcore_barrier()

    return k


def _make_mid(w, cc, kk):
    nch_arr, nch_e = _geom(cc, kk)

    @functools.partial(
        pl.kernel,
        out_type=jax.ShapeDtypeStruct((NC, NPAD, w), jnp.float32),
        mesh=_mesh,
        compiler_params=_sc_params,
        scratch_types=[
            pltpu.VMEM((nch_arr, cc), jnp.int32),
            pltpu.VMEM((nch_arr, cc), jnp.int32),
            [pltpu.VMEM((cc, w), jnp.float32)] * kk,
            pltpu.VMEM_SHARED((NPAD, w), jnp.float32),
            [pltpu.SemaphoreType.DMA] * kk,
        ],
    )
    def k(tbl, src_h, dst_h, zc, out, idxs, idxd, bufs, acc, sems):
        core = lax.axis_index("c")
        sub = lax.axis_index("s")
        nch = nch_e
        _stage_idx(src_h, dst_h, core, sub, idxs, idxd)
        _zero_acc(zc, acc, sub)
        plsc.subcore_barrier()
        _pipe_phase(tbl, idxs, idxd, bufs, sems, acc, nch, kk)
        plsc.subcore_barrier()
        _copy_out(acc, out.at[core], sub)
        plsc.subcore_barrier()

    return k


def _make_l4(cc=128, kk=1):
    d = 128
    nch_arr, nch_e = _geom(cc, kk)

    @functools.partial(
        pl.kernel,
        out_type=jax.ShapeDtypeStruct((NC, 2, NPAD, d), jnp.float32),
        mesh=_mesh,
        compiler_params=_sc_params,
        scratch_types=[
            pltpu.VMEM((nch_arr, cc), jnp.int32),
            pltpu.VMEM((nch_arr, cc), jnp.int32),
            [pltpu.VMEM((cc, d), jnp.float32)] * kk,
            pltpu.VMEM_SHARED((NPAD, d), jnp.float32),
            [pltpu.SemaphoreType.DMA] * kk,
        ],
    )
    def k(t1, t2, src_h, dst_h, zc, out, idxs, idxd, bufs, acc, sems):
        core = lax.axis_index("c")
        sub = lax.axis_index("s")
        nch = nch_e
        _stage_idx(src_h, dst_h, core, sub, idxs, idxd)
        for p, tp in ((0, t1), (1, t2)):
            _zero_acc(zc, acc, sub)
            plsc.subcore_barrier()
            _pipe_phase(tp, idxs, idxd, bufs, sems, acc, nch, kk)
            plsc.subcore_barrier()
            _copy_out(acc, out.at[core, p], sub)
        plsc.subcore_barrier()

    return k


_TCB = 2000


def _dot(a, b):
    return jnp.dot(a, b, preferred_element_type=jnp.float32)


def _l1_body(so, co, x, wp, bp, wn, bn, h, o):
    rcp = 1.0 / (co[...][0, :, 0:1] + 1.0)
    rcn = 1.0 / jnp.maximum(co[...][1, :, 0:1], 1.0)
    xv = x[...]
    agg_p = (so[...][0][:, :3] + xv) * rcp
    agg_n = so[...][1][:, :3] * rcn
    wpv = wp[...]
    wnv = wn[...]
    out_p = _dot(agg_p, wpv[:3]) + _dot(xv, wpv[3:]) + bp[...]
    out_n = _dot(agg_n, wnv[:3]) + _dot(xv, wnv[3:]) + bn[...]
    hv = h[...]
    o[...] = jnp.concatenate(
        [(jnp.maximum(out_p, 0.0) + hv) * 0.5,
         (jnp.maximum(out_n, 0.0) + hv) * 0.5], axis=1)


def _mid_body(d, split, sp, sn, co, z, wp, bp, wn, bn, h, *outs):
    rcp = 1.0 / (co[...][0, :, 0:1] + 1.0)
    rcn = 1.0 / jnp.maximum(co[...][1, :, 0:1], 1.0)
    spv = sp[...]
    snv = sn[...]
    zv = z[...]
    ap1 = (spv[:, :d] + zv[:, :d]) * rcp
    ap2 = (spv[:, d:] + zv[:, d:]) * rcp
    an1 = snv[:, :d] * rcn
    an2 = snv[:, d:] * rcn
    wpv = wp[...]
    wnv = wn[...]
    out_p = (_dot(ap1, wpv[:d]) + _dot(an2, wpv[d:2 * d])
             + _dot(zv[:, :d], wpv[2 * d:]) + bp[...])
    out_n = (_dot(ap2, wnv[:d]) + _dot(an1, wnv[d:2 * d])
             + _dot(zv[:, d:], wnv[2 * d:]) + bn[...])
    hv = h[...]
    zp = (jnp.maximum(out_p, 0.0) + hv) * 0.5
    zn = (jnp.maximum(out_n, 0.0) + hv) * 0.5
    if split:
        outs[0][...] = zp
        outs[1][...] = zn
    else:
        outs[0][...] = jnp.concatenate([zp, zn], axis=1)


def _l4_body(d, s4, co, z1, z2, wp, bp, wn, bn, h, o1, o2):
    rcp = 1.0 / (co[...][0, :, 0:1] + 1.0)
    rcn = 1.0 / jnp.maximum(co[...][1, :, 0:1], 1.0)
    s4v = s4[...]
    ap1 = (s4v[0, 0] + z1[...]) * rcp
    ap2 = (s4v[0, 1] + z2[...]) * rcp
    an1 = s4v[1, 0] * rcn
    an2 = s4v[1, 1] * rcn
    wpv = wp[...]
    wnv = wn[...]
    out_p = (_dot(ap1, wpv[:d]) + _dot(an2, wpv[d:2 * d])
             + _dot(z1[...], wpv[2 * d:]) + bp[...])
    out_n = (_dot(ap2, wnv[:d]) + _dot(an1, wnv[d:2 * d])
             + _dot(z2[...], wnv[2 * d:]) + bn[...])
    hv = h[...]
    o1[...] = (jnp.maximum(out_p, 0.0) + hv) * 0.5
    o2[...] = (jnp.maximum(out_n, 0.0) + hv) * 0.5


def _decq_body(z1, z2, mu, q1, q2):
    muv = mu[...]
    mu2 = jnp.sum(muv * muv, axis=1)[None, :]
    for z, q in ((z1, q1), (z2, q2)):
        zv = z[...]
        z2s = jnp.sum(zv * zv, axis=1, keepdims=True)
        cross = lax.dot_general(zv, muv, (((1,), (1,)), ((), ())),
                                preferred_element_type=jnp.float32)
        d2 = z2s + mu2 - 2.0 * cross
        qv = 1.0 / (1.0 + jnp.maximum(d2, 0.0))
        q[...] = qv / jnp.sum(qv, axis=1, keepdims=True)


def _row_spec(cols):
    return pl.BlockSpec((_TCB, cols), lambda i: (i, 0))


def _full_spec(r, c):
    return pl.BlockSpec((r, c), lambda i: (0, 0))


_cnt_spec = pl.BlockSpec((2, _TCB, 16), lambda i: (0, i, 0), )


def _tc_layer1(s1, x, wp, bp, wn, bn, h):
    g = N // _TCB
    dout = 32
    return pl.pallas_call(
        _l1_body,
        grid=(g,),
        in_specs=[
            pl.BlockSpec((2, None, _TCB, 16), lambda i: (0, 0, i, 0)),
            pl.BlockSpec((2, None, _TCB, 16), lambda i: (0, 1, i, 0)),
            _row_spec(3),
            _full_spec(6, dout), _full_spec(1, dout),
            _full_spec(6, dout), _full_spec(1, dout),
            _row_spec(dout),
        ],
        out_specs=pl.BlockSpec((_TCB, 2 * dout), lambda i: (i, 0)),
        out_shape=jax.ShapeDtypeStruct((N, 2 * dout), jnp.float32),
    )(s1, s1, x, wp, bp, wn, bn, h)


def _tc_mid(d, dout, split, s, s1, z, wp, bp, wn, bn, h):
    g = N // _TCB
    if split:
        out_specs = [_row_spec(dout), _row_spec(dout)]
        out_shape = [jax.ShapeDtypeStruct((N, dout), jnp.float32)] * 2
    else:
        out_specs = pl.BlockSpec((_TCB, 2 * dout), lambda i: (i, 0))
        out_shape = jax.ShapeDtypeStruct((N, 2 * dout), jnp.float32)
    return pl.pallas_call(
        functools.partial(_mid_body, d, split),
        grid=(g,),
        in_specs=[
            pl.BlockSpec((None, _TCB, 2 * d), lambda i: (0, i, 0)),
            pl.BlockSpec((None, _TCB, 2 * d), lambda i: (1, i, 0)),
            pl.BlockSpec((2, None, _TCB, 16), lambda i: (0, 1, i, 0)),
            _row_spec(2 * d),
            _full_spec(3 * d, dout), _full_spec(1, dout),
            _full_spec(3 * d, dout), _full_spec(1, dout),
            _row_spec(dout),
        ],
        out_specs=out_specs,
        out_shape=out_shape,
    )(s, s, s1, z, wp, bp, wn, bn, h)


def _tc_l4(s4, s1, z1, z2, wp, bp, wn, bn, h):
    g = N // _TCB
    d, dout = 128, 256
    return pl.pallas_call(
        functools.partial(_l4_body, d),
        grid=(g,),
        in_specs=[
            pl.BlockSpec((2, 2, _TCB, d), lambda i: (0, 0, i, 0)),
            pl.BlockSpec((2, None, _TCB, 16), lambda i: (0, 1, i, 0)),
            _row_spec(d), _row_spec(d),
            _full_spec(3 * d, dout), _full_spec(1, dout),
            _full_spec(3 * d, dout), _full_spec(1, dout),
            _row_spec(dout),
        ],
        out_specs=[_row_spec(dout), _row_spec(dout)],
        out_shape=[jax.ShapeDtypeStruct((N, dout), jnp.float32)] * 2,
    )(s4, s1, z1, z2, wp, bp, wn, bn, h)


def _tc_decq(z1, z2, mu):
    g = N // _TCB
    d = IN_DIMS[4]
    return pl.pallas_call(
        _decq_body,
        grid=(g,),
        in_specs=[
            _row_spec(d), _row_spec(d),
            _full_spec(N_CLUSTERS, d),
        ],
        out_specs=[_row_spec(N_CLUSTERS), _row_spec(N_CLUSTERS)],
        out_shape=[jax.ShapeDtypeStruct((N, N_CLUSTERS), jnp.float32)] * 2,
    )(z1, z2, mu)


def _pad_edges(src, dst):
    pad = EPAD - src.shape[0]
    src_p = jnp.concatenate([src, jnp.zeros((pad,), jnp.int32)])
    dst_p = jnp.concatenate([dst, jnp.full((pad,), N, jnp.int32)])
    return src_p, dst_p


def _tile_layout(arr, cc):
    return arr.reshape(NW, EPT // cc, cc)


def kernel(x, edge_index_sim, edge_index_disim, h1, h2, h3, h4,
           W1p, b1p, W1n, b1n, W2p, b2p, W2n, b2n, W3p, b3p, W3n, b3n,
           W4p, b4p, W4n, b4n, mu):
    sp_s, dp_s = _pad_edges(edge_index_sim[0], edge_index_sim[1])
    sn_s, dn_s = _pad_edges(edge_index_disim[0], edge_index_disim[1])
    src128 = jnp.stack([_tile_layout(sp_s, 128), _tile_layout(sn_s, 128)])
    dst128 = jnp.stack([_tile_layout(dp_s, 128), _tile_layout(dn_s, 128)])

    x16 = jnp.pad(x, ((0, 0), (0, 13)))
    ones16 = jnp.ones((128, 16), jnp.float32)

    def zc(w):
        return jnp.zeros((ROWS_PT, w), jnp.float32)

    s1 = _make_l1()(x16, src128, dst128, zc(16), ones16)
    z1 = _tc_layer1(s1, x, W1p, b1p.reshape(1, -1), W1n, b1n.reshape(1, -1),
                    h1)

    s2 = _make_mid(64, 128, 1)(z1, src128, dst128, zc(64))
    z2 = _tc_mid(32, 64, False, s2, s1, z1,
                 W2p, b2p.reshape(1, -1), W2n, b2n.reshape(1, -1), h2)

    s3 = _make_mid(128, 128, 1)(z2, src128, dst128, zc(128))
    z3a, z3b = _tc_mid(64, 128, True, s3, s1, z2,
                       W3p, b3p.reshape(1, -1), W3n, b3n.reshape(1, -1), h3)

    s4 = _make_l4()(z3a, z3b, src128, dst128, zc(128))
    z4a, z4b = _tc_l4(s4, s1, z3a, z3b,
                      W4p, b4p.reshape(1, -1), W4n, b4n.reshape(1, -1), h4)

    return _tc_decq(z4a, z4b, mu)


# speedup vs baseline: 1.5030x; 1.0638x over previous
import functools

import jax
import jax.numpy as jnp
from jax import lax
from jax.experimental import pallas as pl
from jax.experimental.pallas import tpu as pltpu
from jax.experimental.pallas import tpu_sc as plsc

N = 10000
E = 320000
EP = E + N
IN_DIMS = [3, 32, 64, 128, 256]
N_CLUSTERS = 30

NC, NS = 2, 16
NW = NC * NS
EPT = 10240
EPAD = EPT * NW
NPAD = 10240
ROWS_PT = NPAD // NS

_mesh = plsc.VectorSubcoreMesh(
    core_axis_name="c", subcore_axis_name="s", num_cores=NC, num_subcores=NS)
_sc_params = pltpu.CompilerParams(use_tc_tiling_on_sc=False)


def _geom(cc, kk):
    nch_arr = EPT // cc

    def bound(real):
        gch = -(-real // cc)
        b = -(-gch // NW)
        return min(-(-b // kk) * kk, nch_arr)

    return nch_arr, bound(E)


def _stage_idx(src_h, dst_h, core, sub, idxs, idxd):
    pltpu.sync_copy(src_h.at[core, sub], idxs)
    pltpu.sync_copy(dst_h.at[core, sub], idxd)


def _zero_acc(zc, acc, sub):
    pltpu.sync_copy(zc, acc.at[pl.ds(sub * ROWS_PT, ROWS_PT)])


def _copy_out(acc, out_ref, sub):
    pltpu.sync_copy(acc.at[pl.ds(sub * ROWS_PT, ROWS_PT)],
                    out_ref.at[pl.ds(sub * ROWS_PT, ROWS_PT)])


def _pipe_phase(tbl, idxs, idxd, bufs, sems, acc, nch, kk):
    @pl.loop(0, nch, step=kk)
    def _(j):
        descs = [
            pltpu.async_copy(tbl.at[idxs.at[j + b]], bufs[b], sems[b])
            for b in range(kk)
        ]
        for b in range(kk):
            descs[b].wait()
            pltpu.sync_copy(bufs[b], acc.at[idxd.at[j + b]], add=True)


def _make_l1(cc=128, kk=2):
    d = 16
    nch_arr, nch_e = _geom(cc, kk)

    @functools.partial(
        pl.kernel,
        out_type=jax.ShapeDtypeStruct((NC, 2, NPAD, d), jnp.float32),
        mesh=_mesh,
        compiler_params=_sc_params,
        scratch_types=[
            pltpu.VMEM((nch_arr, cc), jnp.int32),
            pltpu.VMEM((nch_arr, cc), jnp.int32),
            [pltpu.VMEM((cc, d), jnp.float32)] * kk,
            pltpu.VMEM((cc, d), jnp.float32),
            pltpu.VMEM_SHARED((NPAD, d), jnp.float32),
            [pltpu.SemaphoreType.DMA] * kk,
        ],
    )
    def k(t1, src_h, dst_h, zc, ones_h, out, idxs, idxd, bufs, ones_v, acc,
          sems):
        core = lax.axis_index("c")
        sub = lax.axis_index("s")
        nch = nch_e
        _stage_idx(src_h, dst_h, core, sub, idxs, idxd)
        pltpu.sync_copy(ones_h, ones_v)

        _zero_acc(zc, acc, sub)
        plsc.subcore_barrier()
        _pipe_phase(t1, idxs, idxd, bufs, sems, acc, nch, kk)
        plsc.subcore_barrier()
        _copy_out(acc, out.at[core, 0], sub)

        _zero_acc(zc, acc, sub)
        plsc.subcore_barrier()

        @pl.loop(0, nch)
        def _(j):
            pltpu.sync_copy(ones_v, acc.at[idxd.at[j]], add=True)

        plsc.subcore_barrier()
        _copy_out(acc, out.at[core, 1], sub)
        plsc.subcore_barrier()

    return k


def _make_mid(w, cc, kk):
    nch_arr, nch_e = _geom(cc, kk)

    @functools.partial(
        pl.kernel,
        out_type=jax.ShapeDtypeStruct((NC, NPAD, w), jnp.float32),
        mesh=_mesh,
        compiler_params=_sc_params,
        scratch_types=[
            pltpu.VMEM((nch_arr, cc), jnp.int32),
            pltpu.VMEM((nch_arr, cc), jnp.int32),
            [pltpu.VMEM((cc, w), jnp.float32)] * kk,
            pltpu.VMEM_SHARED((NPAD, w), jnp.float32),
            [pltpu.SemaphoreType.DMA] * kk,
        ],
    )
    def k(tbl, src_h, dst_h, zc, out, idxs, idxd, bufs, acc, sems):
        core = lax.axis_index("c")
        sub = lax.axis_index("s")
        nch = nch_e
        _stage_idx(src_h, dst_h, core, sub, idxs, idxd)
        _zero_acc(zc, acc, sub)
        plsc.subcore_barrier()
        _pipe_phase(tbl, idxs, idxd, bufs, sems, acc, nch, kk)
        plsc.subcore_barrier()
        _copy_out(acc, out.at[core], sub)
        plsc.subcore_barrier()

    return k


def _make_l4(cc=128, kk=1):
    d = 128
    nch_arr, nch_e = _geom(cc, kk)

    @functools.partial(
        pl.kernel,
        out_type=jax.ShapeDtypeStruct((NC, 2, NPAD, d), jnp.float32),
        mesh=_mesh,
        compiler_params=_sc_params,
        scratch_types=[
            pltpu.VMEM((nch_arr, cc), jnp.int32),
            pltpu.VMEM((nch_arr, cc), jnp.int32),
            [pltpu.VMEM((cc, d), jnp.float32)] * kk,
            pltpu.VMEM_SHARED((NPAD, d), jnp.float32),
            [pltpu.SemaphoreType.DMA] * kk,
        ],
    )
    def k(t1, t2, src_h, dst_h, zc, out, idxs, idxd, bufs, acc, sems):
        core = lax.axis_index("c")
        sub = lax.axis_index("s")
        nch = nch_e
        _stage_idx(src_h, dst_h, core, sub, idxs, idxd)
        for p, tp in ((0, t1), (1, t2)):
            _zero_acc(zc, acc, sub)
            plsc.subcore_barrier()
            _pipe_phase(tp, idxs, idxd, bufs, sems, acc, nch, kk)
            plsc.subcore_barrier()
            _copy_out(acc, out.at[core, p], sub)
        plsc.subcore_barrier()

    return k


_TCB = 2000


def _dot(a, b):
    return jnp.dot(a, b, preferred_element_type=jnp.float32)


def _l1_body(so, co, x, wp, bp, wn, bn, h, o):
    rcp = 1.0 / (co[...][0, :, 0:1] + 1.0)
    rcn = 1.0 / jnp.maximum(co[...][1, :, 0:1], 1.0)
    xv = x[...]
    agg_p = (so[...][0][:, :3] + xv) * rcp
    agg_n = so[...][1][:, :3] * rcn
    wpv = wp[...]
    wnv = wn[...]
    out_p = _dot(agg_p, wpv[:3]) + _dot(xv, wpv[3:]) + bp[...]
    out_n = _dot(agg_n, wnv[:3]) + _dot(xv, wnv[3:]) + bn[...]
    hv = h[...]
    o[...] = jnp.concatenate(
        [(jnp.maximum(out_p, 0.0) + hv) * 0.5,
         (jnp.maximum(out_n, 0.0) + hv) * 0.5], axis=1)


def _mid_body(d, split, sp, sn, co, z, wp, bp, wn, bn, h, *outs):
    rcp = 1.0 / (co[...][0, :, 0:1] + 1.0)
    rcn = 1.0 / jnp.maximum(co[...][1, :, 0:1], 1.0)
    spv = sp[...]
    snv = sn[...]
    zv = z[...]
    ap1 = (spv[:, :d] + zv[:, :d]) * rcp
    ap2 = (spv[:, d:] + zv[:, d:]) * rcp
    an1 = snv[:, :d] * rcn
    an2 = snv[:, d:] * rcn
    wpv = wp[...]
    wnv = wn[...]
    out_p = (_dot(ap1, wpv[:d]) + _dot(an2, wpv[d:2 * d])
             + _dot(zv[:, :d], wpv[2 * d:]) + bp[...])
    out_n = (_dot(ap2, wnv[:d]) + _dot(an1, wnv[d:2 * d])
             + _dot(zv[:, d:], wnv[2 * d:]) + bn[...])
    hv = h[...]
    zp = (jnp.maximum(out_p, 0.0) + hv) * 0.5
    zn = (jnp.maximum(out_n, 0.0) + hv) * 0.5
    if split:
        outs[0][...] = zp
        outs[1][...] = zn
    else:
        outs[0][...] = jnp.concatenate([zp, zn], axis=1)


def _l4_body(d, s4, co, z1, z2, wp, bp, wn, bn, h, o1, o2):
    rcp = 1.0 / (co[...][0, :, 0:1] + 1.0)
    rcn = 1.0 / jnp.maximum(co[...][1, :, 0:1], 1.0)
    s4v = s4[...]
    ap1 = (s4v[0, 0] + z1[...]) * rcp
    ap2 = (s4v[0, 1] + z2[...]) * rcp
    an1 = s4v[1, 0] * rcn
    an2 = s4v[1, 1] * rcn
    wpv = wp[...]
    wnv = wn[...]
    out_p = (_dot(ap1, wpv[:d]) + _dot(an2, wpv[d:2 * d])
             + _dot(z1[...], wpv[2 * d:]) + bp[...])
    out_n = (_dot(ap2, wnv[:d]) + _dot(an1, wnv[d:2 * d])
             + _dot(z2[...], wnv[2 * d:]) + bn[...])
    hv = h[...]
    o1[...] = (jnp.maximum(out_p, 0.0) + hv) * 0.5
    o2[...] = (jnp.maximum(out_n, 0.0) + hv) * 0.5


def _decq_body(z1, z2, mu, q1, q2):
    muv = mu[...]
    mu2 = jnp.sum(muv * muv, axis=1)[None, :]
    for z, q in ((z1, q1), (z2, q2)):
        zv = z[...]
        z2s = jnp.sum(zv * zv, axis=1, keepdims=True)
        cross = lax.dot_general(zv, muv, (((1,), (1,)), ((), ())),
                                preferred_element_type=jnp.float32)
        d2 = z2s + mu2 - 2.0 * cross
        qv = 1.0 / (1.0 + jnp.maximum(d2, 0.0))
        q[...] = qv / jnp.sum(qv, axis=1, keepdims=True)


def _row_spec(cols):
    return pl.BlockSpec((_TCB, cols), lambda i: (i, 0))


def _full_spec(r, c):
    return pl.BlockSpec((r, c), lambda i: (0, 0))


_cnt_spec = pl.BlockSpec((2, _TCB, 16), lambda i: (0, i, 0), )


def _tc_layer1(s1, x, wp, bp, wn, bn, h):
    g = N // _TCB
    dout = 32
    return pl.pallas_call(
        _l1_body,
        grid=(g,),
        in_specs=[
            pl.BlockSpec((2, None, _TCB, 16), lambda i: (0, 0, i, 0)),
            pl.BlockSpec((2, None, _TCB, 16), lambda i: (0, 1, i, 0)),
            _row_spec(3),
            _full_spec(6, dout), _full_spec(1, dout),
            _full_spec(6, dout), _full_spec(1, dout),
            _row_spec(dout),
        ],
        out_specs=pl.BlockSpec((_TCB, 2 * dout), lambda i: (i, 0)),
        out_shape=jax.ShapeDtypeStruct((N, 2 * dout), jnp.float32),
    )(s1, s1, x, wp, bp, wn, bn, h)


def _tc_mid(d, dout, split, s, s1, z, wp, bp, wn, bn, h):
    g = N // _TCB
    if split:
        out_specs = [_row_spec(dout), _row_spec(dout)]
        out_shape = [jax.ShapeDtypeStruct((N, dout), jnp.float32)] * 2
    else:
        out_specs = pl.BlockSpec((_TCB, 2 * dout), lambda i: (i, 0))
        out_shape = jax.ShapeDtypeStruct((N, 2 * dout), jnp.float32)
    return pl.pallas_call(
        functools.partial(_mid_body, d, split),
        grid=(g,),
        in_specs=[
            pl.BlockSpec((None, _TCB, 2 * d), lambda i: (0, i, 0)),
            pl.BlockSpec((None, _TCB, 2 * d), lambda i: (1, i, 0)),
            pl.BlockSpec((2, None, _TCB, 16), lambda i: (0, 1, i, 0)),
            _row_spec(2 * d),
            _full_spec(3 * d, dout), _full_spec(1, dout),
            _full_spec(3 * d, dout), _full_spec(1, dout),
            _row_spec(dout),
        ],
        out_specs=out_specs,
        out_shape=out_shape,
    )(s, s, s1, z, wp, bp, wn, bn, h)


def _tc_l4(s4, s1, z1, z2, wp, bp, wn, bn, h):
    g = N // _TCB
    d, dout = 128, 256
    return pl.pallas_call(
        functools.partial(_l4_body, d),
        grid=(g,),
        in_specs=[
            pl.BlockSpec((2, 2, _TCB, d), lambda i: (0, 0, i, 0)),
            pl.BlockSpec((2, None, _TCB, 16), lambda i: (0, 1, i, 0)),
            _row_spec(d), _row_spec(d),
            _full_spec(3 * d, dout), _full_spec(1, dout),
            _full_spec(3 * d, dout), _full_spec(1, dout),
            _row_spec(dout),
        ],
        out_specs=[_row_spec(dout), _row_spec(dout)],
        out_shape=[jax.ShapeDtypeStruct((N, dout), jnp.float32)] * 2,
    )(s4, s1, z1, z2, wp, bp, wn, bn, h)


def _tc_decq(z1, z2, mu):
    g = N // _TCB
    d = IN_DIMS[4]
    return pl.pallas_call(
        _decq_body,
        grid=(g,),
        in_specs=[
            _row_spec(d), _row_spec(d),
            _full_spec(N_CLUSTERS, d),
        ],
        out_specs=[_row_spec(N_CLUSTERS), _row_spec(N_CLUSTERS)],
        out_shape=[jax.ShapeDtypeStruct((N, N_CLUSTERS), jnp.float32)] * 2,
    )(z1, z2, mu)


def _pad_edges(src, dst):
    pad = EPAD - src.shape[0]
    src_p = jnp.concatenate([src, jnp.zeros((pad,), jnp.int32)])
    dst_p = jnp.concatenate([dst, jnp.full((pad,), N, jnp.int32)])
    return src_p, dst_p


def _tile_layout(arr, cc):
    return arr.reshape(NW, EPT // cc, cc)


def kernel(x, edge_index_sim, edge_index_disim, h1, h2, h3, h4,
           W1p, b1p, W1n, b1n, W2p, b2p, W2n, b2n, W3p, b3p, W3n, b3n,
           W4p, b4p, W4n, b4n, mu):
    sp_s, dp_s = _pad_edges(edge_index_sim[0], edge_index_sim[1])
    sn_s, dn_s = _pad_edges(edge_index_disim[0], edge_index_disim[1])
    src128 = jnp.stack([_tile_layout(sp_s, 128), _tile_layout(sn_s, 128)])
    dst128 = jnp.stack([_tile_layout(dp_s, 128), _tile_layout(dn_s, 128)])

    x16 = jnp.pad(x, ((0, 0), (0, 13)))
    ones16 = jnp.ones((128, 16), jnp.float32)

    def zc(w):
        return jnp.zeros((ROWS_PT, w), jnp.float32)

    s1 = _make_l1()(x16, src128, dst128, zc(16), ones16)
    z1 = _tc_layer1(s1, x, W1p, b1p.reshape(1, -1), W1n, b1n.reshape(1, -1),
                    h1)

    s2 = _make_mid(64, 128, 2)(z1, src128, dst128, zc(64))
    z2 = _tc_mid(32, 64, False, s2, s1, z1,
                 W2p, b2p.reshape(1, -1), W2n, b2n.reshape(1, -1), h2)

    s3 = _make_mid(128, 128, 1)(z2, src128, dst128, zc(128))
    z3a, z3b = _tc_mid(64, 128, True, s3, s1, z2,
                       W3p, b3p.reshape(1, -1), W3n, b3n.reshape(1, -1), h3)

    s4 = _make_l4()(z3a, z3b, src128, dst128, zc(128))
    z4a, z4b = _tc_l4(s4, s1, z3a, z3b,
                      W4p, b4p.reshape(1, -1), W4n, b4n.reshape(1, -1), h4)

    return _tc_decq(z4a, z4b, mu)


# speedup vs baseline: 1.5762x; 1.0487x over previous
import functools

import jax
import jax.numpy as jnp
from jax import lax
from jax.experimental import pallas as pl
from jax.experimental.pallas import tpu as pltpu
from jax.experimental.pallas import tpu_sc as plsc

N = 10000
E = 320000
EP = E + N
IN_DIMS = [3, 32, 64, 128, 256]
N_CLUSTERS = 30

NC, NS = 2, 16
NW = NC * NS
EPT = 10240
EPAD = EPT * NW
NPAD = 10240
ROWS_PT = NPAD // NS

_mesh = plsc.VectorSubcoreMesh(
    core_axis_name="c", subcore_axis_name="s", num_cores=NC, num_subcores=NS)
_sc_params = pltpu.CompilerParams(use_tc_tiling_on_sc=False)


def _geom(cc, kk):
    nch_arr = EPT // cc

    def bound(real):
        gch = -(-real // cc)
        b = -(-gch // NW)
        return min(-(-b // kk) * kk, nch_arr)

    return nch_arr, bound(E)


def _stage_idx(src_h, dst_h, core, sub, idxs, idxd):
    pltpu.sync_copy(src_h.at[core, sub], idxs)
    pltpu.sync_copy(dst_h.at[core, sub], idxd)


def _zero_acc(zc, acc, sub):
    pltpu.sync_copy(zc, acc.at[pl.ds(sub * ROWS_PT, ROWS_PT)])


def _copy_out(acc, out_ref, sub):
    pltpu.sync_copy(acc.at[pl.ds(sub * ROWS_PT, ROWS_PT)],
                    out_ref.at[pl.ds(sub * ROWS_PT, ROWS_PT)])


def _pipe_phase(tbl, idxs, idxd, bufs, sems, acc, nch, kk):
    @pl.loop(0, nch, step=kk)
    def _(j):
        descs = [
            pltpu.async_copy(tbl.at[idxs.at[j + b]], bufs[b], sems[b])
            for b in range(kk)
        ]
        for b in range(kk):
            descs[b].wait()
            pltpu.sync_copy(bufs[b], acc.at[idxd.at[j + b]], add=True)


def _make_l1(cc=128, kk=2):
    d = 16
    nch_arr, nch_e = _geom(cc, kk)

    @functools.partial(
        pl.kernel,
        out_type=jax.ShapeDtypeStruct((NC, 2, NPAD, d), jnp.float32),
        mesh=_mesh,
        compiler_params=_sc_params,
        scratch_types=[
            pltpu.VMEM((nch_arr, cc), jnp.int32),
            pltpu.VMEM((nch_arr, cc), jnp.int32),
            [pltpu.VMEM((cc, d), jnp.float32)] * kk,
            pltpu.VMEM((cc, d), jnp.float32),
            pltpu.VMEM_SHARED((NPAD, d), jnp.float32),
            [pltpu.SemaphoreType.DMA] * kk,
        ],
    )
    def k(t1, src_h, dst_h, zc, ones_h, out, idxs, idxd, bufs, ones_v, acc,
          sems):
        core = lax.axis_index("c")
        sub = lax.axis_index("s")
        nch = nch_e
        _stage_idx(src_h, dst_h, core, sub, idxs, idxd)
        pltpu.sync_copy(ones_h, ones_v)

        _zero_acc(zc, acc, sub)
        plsc.subcore_barrier()
        _pipe_phase(t1, idxs, idxd, bufs, sems, acc, nch, kk)
        plsc.subcore_barrier()
        _copy_out(acc, out.at[core, 0], sub)

        _zero_acc(zc, acc, sub)
        plsc.subcore_barrier()

        @pl.loop(0, nch)
        def _(j):
            pltpu.sync_copy(ones_v, acc.at[idxd.at[j]], add=True)

        plsc.subcore_barrier()
        _copy_out(acc, out.at[core, 1], sub)
        plsc.subcore_barrier()

    return k


def _make_mid(w, cc, kk):
    nch_arr, nch_e = _geom(cc, kk)

    @functools.partial(
        pl.kernel,
        out_type=jax.ShapeDtypeStruct((NC, NPAD, w), jnp.float32),
        mesh=_mesh,
        compiler_params=_sc_params,
        scratch_types=[
            pltpu.VMEM((nch_arr, cc), jnp.int32),
            pltpu.VMEM((nch_arr, cc), jnp.int32),
            [pltpu.VMEM((cc, w), jnp.float32)] * kk,
            pltpu.VMEM_SHARED((NPAD, w), jnp.float32),
            [pltpu.SemaphoreType.DMA] * kk,
        ],
    )
    def k(tbl, src_h, dst_h, zc, out, idxs, idxd, bufs, acc, sems):
        core = lax.axis_index("c")
        sub = lax.axis_index("s")
        nch = nch_e
        _stage_idx(src_h, dst_h, core, sub, idxs, idxd)
        _zero_acc(zc, acc, sub)
        plsc.subcore_barrier()
        _pipe_phase(tbl, idxs, idxd, bufs, sems, acc, nch, kk)
        plsc.subcore_barrier()
        _copy_out(acc, out.at[core], sub)
        plsc.subcore_barrier()

    return k


def _make_l4(cc=80, kk=2):
    d = 128
    nch_arr, nch_e = _geom(cc, kk)

    @functools.partial(
        pl.kernel,
        out_type=jax.ShapeDtypeStruct((NC, 2, NPAD, d), jnp.float32),
        mesh=_mesh,
        compiler_params=_sc_params,
        scratch_types=[
            pltpu.VMEM((nch_arr, cc), jnp.int32),
            pltpu.VMEM((nch_arr, cc), jnp.int32),
            [pltpu.VMEM((cc, d), jnp.float32)] * kk,
            pltpu.VMEM_SHARED((NPAD, d), jnp.float32),
            [pltpu.SemaphoreType.DMA] * kk,
        ],
    )
    def k(t1, t2, src_h, dst_h, zc, out, idxs, idxd, bufs, acc, sems):
        core = lax.axis_index("c")
        sub = lax.axis_index("s")
        nch = nch_e
        _stage_idx(src_h, dst_h, core, sub, idxs, idxd)
        for p, tp in ((0, t1), (1, t2)):
            _zero_acc(zc, acc, sub)
            plsc.subcore_barrier()
            _pipe_phase(tp, idxs, idxd, bufs, sems, acc, nch, kk)
            plsc.subcore_barrier()
            _copy_out(acc, out.at[core, p], sub)
        plsc.subcore_barrier()

    return k


_TCB = 2000


def _dot(a, b):
    return jnp.dot(a, b, preferred_element_type=jnp.float32)


def _l1_body(so, co, x, wp, bp, wn, bn, h, o):
    rcp = 1.0 / (co[...][0, :, 0:1] + 1.0)
    rcn = 1.0 / jnp.maximum(co[...][1, :, 0:1], 1.0)
    xv = x[...]
    agg_p = (so[...][0][:, :3] + xv) * rcp
    agg_n = so[...][1][:, :3] * rcn
    wpv = wp[...]
    wnv = wn[...]
    out_p = _dot(agg_p, wpv[:3]) + _dot(xv, wpv[3:]) + bp[...]
    out_n = _dot(agg_n, wnv[:3]) + _dot(xv, wnv[3:]) + bn[...]
    hv = h[...]
    o[...] = jnp.concatenate(
        [(jnp.maximum(out_p, 0.0) + hv) * 0.5,
         (jnp.maximum(out_n, 0.0) + hv) * 0.5], axis=1)


def _mid_body(d, split, sp, sn, co, z, wp, bp, wn, bn, h, *outs):
    rcp = 1.0 / (co[...][0, :, 0:1] + 1.0)
    rcn = 1.0 / jnp.maximum(co[...][1, :, 0:1], 1.0)
    spv = sp[...]
    snv = sn[...]
    zv = z[...]
    ap1 = (spv[:, :d] + zv[:, :d]) * rcp
    ap2 = (spv[:, d:] + zv[:, d:]) * rcp
    an1 = snv[:, :d] * rcn
    an2 = snv[:, d:] * rcn
    wpv = wp[...]
    wnv = wn[...]
    out_p = (_dot(ap1, wpv[:d]) + _dot(an2, wpv[d:2 * d])
             + _dot(zv[:, :d], wpv[2 * d:]) + bp[...])
    out_n = (_dot(ap2, wnv[:d]) + _dot(an1, wnv[d:2 * d])
             + _dot(zv[:, d:], wnv[2 * d:]) + bn[...])
    hv = h[...]
    zp = (jnp.maximum(out_p, 0.0) + hv) * 0.5
    zn = (jnp.maximum(out_n, 0.0) + hv) * 0.5
    if split:
        outs[0][...] = zp
        outs[1][...] = zn
    else:
        outs[0][...] = jnp.concatenate([zp, zn], axis=1)


def _l4_body(d, s4, co, z1, z2, wp, bp, wn, bn, h, o1, o2):
    rcp = 1.0 / (co[...][0, :, 0:1] + 1.0)
    rcn = 1.0 / jnp.maximum(co[...][1, :, 0:1], 1.0)
    s4v = s4[...]
    ap1 = (s4v[0, 0] + z1[...]) * rcp
    ap2 = (s4v[0, 1] + z2[...]) * rcp
    an1 = s4v[1, 0] * rcn
    an2 = s4v[1, 1] * rcn
    wpv = wp[...]
    wnv = wn[...]
    out_p = (_dot(ap1, wpv[:d]) + _dot(an2, wpv[d:2 * d])
             + _dot(z1[...], wpv[2 * d:]) + bp[...])
    out_n = (_dot(ap2, wnv[:d]) + _dot(an1, wnv[d:2 * d])
             + _dot(z2[...], wnv[2 * d:]) + bn[...])
    hv = h[...]
    o1[...] = (jnp.maximum(out_p, 0.0) + hv) * 0.5
    o2[...] = (jnp.maximum(out_n, 0.0) + hv) * 0.5


def _decq_body(z1, z2, mu, q1, q2):
    muv = mu[...]
    mu2 = jnp.sum(muv * muv, axis=1)[None, :]
    for z, q in ((z1, q1), (z2, q2)):
        zv = z[...]
        z2s = jnp.sum(zv * zv, axis=1, keepdims=True)
        cross = lax.dot_general(zv, muv, (((1,), (1,)), ((), ())),
                                preferred_element_type=jnp.float32)
        d2 = z2s + mu2 - 2.0 * cross
        qv = 1.0 / (1.0 + jnp.maximum(d2, 0.0))
        q[...] = qv / jnp.sum(qv, axis=1, keepdims=True)


def _row_spec(cols):
    return pl.BlockSpec((_TCB, cols), lambda i: (i, 0))


def _full_spec(r, c):
    return pl.BlockSpec((r, c), lambda i: (0, 0))


_cnt_spec = pl.BlockSpec((2, _TCB, 16), lambda i: (0, i, 0), )


def _tc_layer1(s1, x, wp, bp, wn, bn, h):
    g = N // _TCB
    dout = 32
    return pl.pallas_call(
        _l1_body,
        grid=(g,),
        in_specs=[
            pl.BlockSpec((2, None, _TCB, 16), lambda i: (0, 0, i, 0)),
            pl.BlockSpec((2, None, _TCB, 16), lambda i: (0, 1, i, 0)),
            _row_spec(3),
            _full_spec(6, dout), _full_spec(1, dout),
            _full_spec(6, dout), _full_spec(1, dout),
            _row_spec(dout),
        ],
        out_specs=pl.BlockSpec((_TCB, 2 * dout), lambda i: (i, 0)),
        out_shape=jax.ShapeDtypeStruct((N, 2 * dout), jnp.float32),
    )(s1, s1, x, wp, bp, wn, bn, h)


def _tc_mid(d, dout, split, s, s1, z, wp, bp, wn, bn, h):
    g = N // _TCB
    if split:
        out_specs = [_row_spec(dout), _row_spec(dout)]
        out_shape = [jax.ShapeDtypeStruct((N, dout), jnp.float32)] * 2
    else:
        out_specs = pl.BlockSpec((_TCB, 2 * dout), lambda i: (i, 0))
        out_shape = jax.ShapeDtypeStruct((N, 2 * dout), jnp.float32)
    return pl.pallas_call(
        functools.partial(_mid_body, d, split),
        grid=(g,),
        in_specs=[
            pl.BlockSpec((None, _TCB, 2 * d), lambda i: (0, i, 0)),
            pl.BlockSpec((None, _TCB, 2 * d), lambda i: (1, i, 0)),
            pl.BlockSpec((2, None, _TCB, 16), lambda i: (0, 1, i, 0)),
            _row_spec(2 * d),
            _full_spec(3 * d, dout), _full_spec(1, dout),
            _full_spec(3 * d, dout), _full_spec(1, dout),
            _row_spec(dout),
        ],
        out_specs=out_specs,
        out_shape=out_shape,
    )(s, s, s1, z, wp, bp, wn, bn, h)


def _tc_l4(s4, s1, z1, z2, wp, bp, wn, bn, h):
    g = N // _TCB
    d, dout = 128, 256
    return pl.pallas_call(
        functools.partial(_l4_body, d),
        grid=(g,),
        in_specs=[
            pl.BlockSpec((2, 2, _TCB, d), lambda i: (0, 0, i, 0)),
            pl.BlockSpec((2, None, _TCB, 16), lambda i: (0, 1, i, 0)),
            _row_spec(d), _row_spec(d),
            _full_spec(3 * d, dout), _full_spec(1, dout),
            _full_spec(3 * d, dout), _full_spec(1, dout),
            _row_spec(dout),
        ],
        out_specs=[_row_spec(dout), _row_spec(dout)],
        out_shape=[jax.ShapeDtypeStruct((N, dout), jnp.float32)] * 2,
    )(s4, s1, z1, z2, wp, bp, wn, bn, h)


def _tc_decq(z1, z2, mu):
    g = N // _TCB
    d = IN_DIMS[4]
    return pl.pallas_call(
        _decq_body,
        grid=(g,),
        in_specs=[
            _row_spec(d), _row_spec(d),
            _full_spec(N_CLUSTERS, d),
        ],
        out_specs=[_row_spec(N_CLUSTERS), _row_spec(N_CLUSTERS)],
        out_shape=[jax.ShapeDtypeStruct((N, N_CLUSTERS), jnp.float32)] * 2,
    )(z1, z2, mu)


def _pad_edges(src, dst):
    pad = EPAD - src.shape[0]
    src_p = jnp.concatenate([src, jnp.zeros((pad,), jnp.int32)])
    dst_p = jnp.concatenate([dst, jnp.full((pad,), N, jnp.int32)])
    return src_p, dst_p


def _tile_layout(arr, cc):
    return arr.reshape(NW, EPT // cc, cc)


def kernel(x, edge_index_sim, edge_index_disim, h1, h2, h3, h4,
           W1p, b1p, W1n, b1n, W2p, b2p, W2n, b2n, W3p, b3p, W3n, b3n,
           W4p, b4p, W4n, b4n, mu):
    sp_s, dp_s = _pad_edges(edge_index_sim[0], edge_index_sim[1])
    sn_s, dn_s = _pad_edges(edge_index_disim[0], edge_index_disim[1])
    src128 = jnp.stack([_tile_layout(sp_s, 128), _tile_layout(sn_s, 128)])
    dst128 = jnp.stack([_tile_layout(dp_s, 128), _tile_layout(dn_s, 128)])
    src80 = jnp.stack([_tile_layout(sp_s, 80), _tile_layout(sn_s, 80)])
    dst80 = jnp.stack([_tile_layout(dp_s, 80), _tile_layout(dn_s, 80)])

    x16 = jnp.pad(x, ((0, 0), (0, 13)))
    ones16 = jnp.ones((128, 16), jnp.float32)

    def zc(w):
        return jnp.zeros((ROWS_PT, w), jnp.float32)

    s1 = _make_l1()(x16, src128, dst128, zc(16), ones16)
    z1 = _tc_layer1(s1, x, W1p, b1p.reshape(1, -1), W1n, b1n.reshape(1, -1),
                    h1)

    s2 = _make_mid(64, 128, 2)(z1, src128, dst128, zc(64))
    z2 = _tc_mid(32, 64, False, s2, s1, z1,
                 W2p, b2p.reshape(1, -1), W2n, b2n.reshape(1, -1), h2)

    s3 = _make_mid(128, 80, 2)(z2, src80, dst80, zc(128))
    z3a, z3b = _tc_mid(64, 128, True, s3, s1, z2,
                       W3p, b3p.reshape(1, -1), W3n, b3n.reshape(1, -1), h3)

    s4 = _make_l4()(z3a, z3b, src80, dst80, zc(128))
    z4a, z4b = _tc_l4(s4, s1, z3a, z3b,
                      W4p, b4p.reshape(1, -1), W4n, b4n.reshape(1, -1), h4)

    return _tc_decq(z4a, z4b, mu)


# speedup vs baseline: 1.5959x; 1.0125x over previous
import functools

import jax
import jax.numpy as jnp
from jax import lax
from jax.experimental import pallas as pl
from jax.experimental.pallas import tpu as pltpu
from jax.experimental.pallas import tpu_sc as plsc

N = 10000
E = 320000
EP = E + N
IN_DIMS = [3, 32, 64, 128, 256]
N_CLUSTERS = 30

NC, NS = 2, 16
NW = NC * NS
EPT = 10240
EPAD = EPT * NW
NPAD = 10240
ROWS_PT = NPAD // NS

_mesh = plsc.VectorSubcoreMesh(
    core_axis_name="c", subcore_axis_name="s", num_cores=NC, num_subcores=NS)
_sc_params = pltpu.CompilerParams(use_tc_tiling_on_sc=False)


def _geom(cc, kk):
    nch_arr = EPT // cc

    def bound(real):
        gch = -(-real // cc)
        b = -(-gch // NW)
        return min(-(-b // kk) * kk, nch_arr)

    return nch_arr, bound(E)


def _stage_idx(src_h, dst_h, core, sub, idxs, idxd):
    pltpu.sync_copy(src_h.at[core, sub], idxs)
    pltpu.sync_copy(dst_h.at[core, sub], idxd)


def _zero_acc(zc, acc, sub):
    pltpu.sync_copy(zc, acc.at[pl.ds(sub * ROWS_PT, ROWS_PT)])


def _copy_out(acc, out_ref, sub):
    pltpu.sync_copy(acc.at[pl.ds(sub * ROWS_PT, ROWS_PT)],
                    out_ref.at[pl.ds(sub * ROWS_PT, ROWS_PT)])


def _pipe_phase(tbl, idxs, idxd, bufs, sems, acc, nch, kk):
    @pl.loop(0, nch, step=kk)
    def _(j):
        descs = [
            pltpu.async_copy(tbl.at[idxs.at[j + b]], bufs[b], sems[b])
            for b in range(kk)
        ]
        for b in range(kk):
            descs[b].wait()
            pltpu.sync_copy(bufs[b], acc.at[idxd.at[j + b]], add=True)


def _make_mid(w, cc, kk):
    nch_arr, nch_e = _geom(cc, kk)

    @functools.partial(
        pl.kernel,
        out_type=jax.ShapeDtypeStruct((NC, NPAD, w), jnp.float32),
        mesh=_mesh,
        compiler_params=_sc_params,
        scratch_types=[
            pltpu.VMEM((nch_arr, cc), jnp.int32),
            pltpu.VMEM((nch_arr, cc), jnp.int32),
            [pltpu.VMEM((cc, w), jnp.float32)] * kk,
            pltpu.VMEM_SHARED((NPAD, w), jnp.float32),
            [pltpu.SemaphoreType.DMA] * kk,
        ],
    )
    def k(tbl, src_h, dst_h, zc, out, idxs, idxd, bufs, acc, sems):
        core = lax.axis_index("c")
        sub = lax.axis_index("s")
        nch = nch_e
        _stage_idx(src_h, dst_h, core, sub, idxs, idxd)
        _zero_acc(zc, acc, sub)
        plsc.subcore_barrier()
        _pipe_phase(tbl, idxs, idxd, bufs, sems, acc, nch, kk)
        plsc.subcore_barrier()
        _copy_out(acc, out.at[core], sub)
        plsc.subcore_barrier()

    return k


def _make_l4(cc=80, kk=2):
    d = 128
    nch_arr, nch_e = _geom(cc, kk)

    @functools.partial(
        pl.kernel,
        out_type=jax.ShapeDtypeStruct((NC, 2, NPAD, d), jnp.float32),
        mesh=_mesh,
        compiler_params=_sc_params,
        scratch_types=[
            pltpu.VMEM((nch_arr, cc), jnp.int32),
            pltpu.VMEM((nch_arr, cc), jnp.int32),
            [pltpu.VMEM((cc, d), jnp.float32)] * kk,
            pltpu.VMEM_SHARED((NPAD, d), jnp.float32),
            [pltpu.SemaphoreType.DMA] * kk,
        ],
    )
    def k(t1, t2, src_h, dst_h, zc, out, idxs, idxd, bufs, acc, sems):
        core = lax.axis_index("c")
        sub = lax.axis_index("s")
        nch = nch_e
        _stage_idx(src_h, dst_h, core, sub, idxs, idxd)
        for p, tp in ((0, t1), (1, t2)):
            _zero_acc(zc, acc, sub)
            plsc.subcore_barrier()
            _pipe_phase(tp, idxs, idxd, bufs, sems, acc, nch, kk)
            plsc.subcore_barrier()
            _copy_out(acc, out.at[core, p], sub)
        plsc.subcore_barrier()

    return k


_TCB = 2000


def _dot(a, b):
    return jnp.dot(a, b, preferred_element_type=jnp.float32)


def _l1_body(so, x, wp, bp, wn, bn, h, o):
    sov = so[...]
    rcp = 1.0 / (sov[0, :, 3:4] + 1.0)
    rcn = 1.0 / jnp.maximum(sov[1, :, 3:4], 1.0)
    xv = x[...]
    agg_p = (sov[0, :, :3] + xv) * rcp
    agg_n = sov[1, :, :3] * rcn
    wpv = wp[...]
    wnv = wn[...]
    out_p = _dot(agg_p, wpv[:3]) + _dot(xv, wpv[3:]) + bp[...]
    out_n = _dot(agg_n, wnv[:3]) + _dot(xv, wnv[3:]) + bn[...]
    hv = h[...]
    o[...] = jnp.concatenate(
        [(jnp.maximum(out_p, 0.0) + hv) * 0.5,
         (jnp.maximum(out_n, 0.0) + hv) * 0.5], axis=1)


def _mid_body(d, split, sp, sn, co, z, wp, bp, wn, bn, h, *outs):
    rcp = 1.0 / (co[...][0, :, 3:4] + 1.0)
    rcn = 1.0 / jnp.maximum(co[...][1, :, 3:4], 1.0)
    spv = sp[...]
    snv = sn[...]
    zv = z[...]
    ap1 = (spv[:, :d] + zv[:, :d]) * rcp
    ap2 = (spv[:, d:] + zv[:, d:]) * rcp
    an1 = snv[:, :d] * rcn
    an2 = snv[:, d:] * rcn
    wpv = wp[...]
    wnv = wn[...]
    out_p = (_dot(ap1, wpv[:d]) + _dot(an2, wpv[d:2 * d])
             + _dot(zv[:, :d], wpv[2 * d:]) + bp[...])
    out_n = (_dot(ap2, wnv[:d]) + _dot(an1, wnv[d:2 * d])
             + _dot(zv[:, d:], wnv[2 * d:]) + bn[...])
    hv = h[...]
    zp = (jnp.maximum(out_p, 0.0) + hv) * 0.5
    zn = (jnp.maximum(out_n, 0.0) + hv) * 0.5
    if split:
        outs[0][...] = zp
        outs[1][...] = zn
    else:
        outs[0][...] = jnp.concatenate([zp, zn], axis=1)


def _l4q_body(d, s4, co, z1, z2, wp, bp, wn, bn, h, mu, q1, q2):
    rcp = 1.0 / (co[...][0, :, 3:4] + 1.0)
    rcn = 1.0 / jnp.maximum(co[...][1, :, 3:4], 1.0)
    s4v = s4[...]
    ap1 = (s4v[0, 0] + z1[...]) * rcp
    ap2 = (s4v[0, 1] + z2[...]) * rcp
    an1 = s4v[1, 0] * rcn
    an2 = s4v[1, 1] * rcn
    wpv = wp[...]
    wnv = wn[...]
    out_p = (_dot(ap1, wpv[:d]) + _dot(an2, wpv[d:2 * d])
             + _dot(z1[...], wpv[2 * d:]) + bp[...])
    out_n = (_dot(ap2, wnv[:d]) + _dot(an1, wnv[d:2 * d])
             + _dot(z2[...], wnv[2 * d:]) + bn[...])
    hv = h[...]
    zp = (jnp.maximum(out_p, 0.0) + hv) * 0.5
    zn = (jnp.maximum(out_n, 0.0) + hv) * 0.5
    muv = mu[...]
    mu2 = jnp.sum(muv * muv, axis=1)[None, :]
    for zv, q in ((zp, q1), (zn, q2)):
        z2s = jnp.sum(zv * zv, axis=1, keepdims=True)
        cross = lax.dot_general(zv, muv, (((1,), (1,)), ((), ())),
                                preferred_element_type=jnp.float32)
        d2 = z2s + mu2 - 2.0 * cross
        qv = 1.0 / (1.0 + jnp.maximum(d2, 0.0))
        q[...] = qv / jnp.sum(qv, axis=1, keepdims=True)


def _row_spec(cols):
    return pl.BlockSpec((_TCB, cols), lambda i: (i, 0))


def _full_spec(r, c):
    return pl.BlockSpec((r, c), lambda i: (0, 0))


_s1_spec = pl.BlockSpec((2, _TCB, 16), lambda i: (0, i, 0))


def _tc_layer1(s1, x, wp, bp, wn, bn, h):
    g = N // _TCB
    dout = 32
    return pl.pallas_call(
        _l1_body,
        grid=(g,),
        in_specs=[
            _s1_spec,
            _row_spec(3),
            _full_spec(6, dout), _full_spec(1, dout),
            _full_spec(6, dout), _full_spec(1, dout),
            _row_spec(dout),
        ],
        out_specs=pl.BlockSpec((_TCB, 2 * dout), lambda i: (i, 0)),
        out_shape=jax.ShapeDtypeStruct((N, 2 * dout), jnp.float32),
    )(s1, x, wp, bp, wn, bn, h)


def _tc_mid(d, dout, split, s, s1, z, wp, bp, wn, bn, h):
    g = N // _TCB
    if split:
        out_specs = [_row_spec(dout), _row_spec(dout)]
        out_shape = [jax.ShapeDtypeStruct((N, dout), jnp.float32)] * 2
    else:
        out_specs = pl.BlockSpec((_TCB, 2 * dout), lambda i: (i, 0))
        out_shape = jax.ShapeDtypeStruct((N, 2 * dout), jnp.float32)
    return pl.pallas_call(
        functools.partial(_mid_body, d, split),
        grid=(g,),
        in_specs=[
            pl.BlockSpec((None, _TCB, 2 * d), lambda i: (0, i, 0)),
            pl.BlockSpec((None, _TCB, 2 * d), lambda i: (1, i, 0)),
            _s1_spec,
            _row_spec(2 * d),
            _full_spec(3 * d, dout), _full_spec(1, dout),
            _full_spec(3 * d, dout), _full_spec(1, dout),
            _row_spec(dout),
        ],
        out_specs=out_specs,
        out_shape=out_shape,
    )(s, s, s1, z, wp, bp, wn, bn, h)


def _tc_l4q(s4, s1, z1, z2, wp, bp, wn, bn, h, mu):
    g = N // _TCB
    d, dout = 128, 256
    return pl.pallas_call(
        functools.partial(_l4q_body, d),
        grid=(g,),
        in_specs=[
            pl.BlockSpec((2, 2, _TCB, d), lambda i: (0, 0, i, 0)),
            _s1_spec,
            _row_spec(d), _row_spec(d),
            _full_spec(3 * d, dout), _full_spec(1, dout),
            _full_spec(3 * d, dout), _full_spec(1, dout),
            _row_spec(dout),
            _full_spec(N_CLUSTERS, dout),
        ],
        out_specs=[_row_spec(N_CLUSTERS), _row_spec(N_CLUSTERS)],
        out_shape=[jax.ShapeDtypeStruct((N, N_CLUSTERS), jnp.float32)] * 2,
    )(s4, s1, z1, z2, wp, bp, wn, bn, h, mu)


def _pad_edges(src, dst):
    pad = EPAD - src.shape[0]
    src_p = jnp.concatenate([src, jnp.zeros((pad,), jnp.int32)])
    dst_p = jnp.concatenate([dst, jnp.full((pad,), N, jnp.int32)])
    return src_p, dst_p


def _tile_layout(arr, cc):
    return arr.reshape(NW, EPT // cc, cc)


def kernel(x, edge_index_sim, edge_index_disim, h1, h2, h3, h4,
           W1p, b1p, W1n, b1n, W2p, b2p, W2n, b2n, W3p, b3p, W3n, b3n,
           W4p, b4p, W4n, b4n, mu):
    sp_s, dp_s = _pad_edges(edge_index_sim[0], edge_index_sim[1])
    sn_s, dn_s = _pad_edges(edge_index_disim[0], edge_index_disim[1])
    src128 = jnp.stack([_tile_layout(sp_s, 128), _tile_layout(sn_s, 128)])
    dst128 = jnp.stack([_tile_layout(dp_s, 128), _tile_layout(dn_s, 128)])
    src80 = jnp.stack([_tile_layout(sp_s, 80), _tile_layout(sn_s, 80)])
    dst80 = jnp.stack([_tile_layout(dp_s, 80), _tile_layout(dn_s, 80)])

    x16 = jnp.pad(x, ((0, 0), (0, 13))).at[:, 3].set(1.0)

    def zc(w):
        return jnp.zeros((ROWS_PT, w), jnp.float32)

    s1 = _make_mid(16, 128, 2)(x16, src128, dst128, zc(16))
    z1 = _tc_layer1(s1, x, W1p, b1p.reshape(1, -1), W1n, b1n.reshape(1, -1),
                    h1)

    s2 = _make_mid(64, 128, 2)(z1, src128, dst128, zc(64))
    z2 = _tc_mid(32, 64, False, s2, s1, z1,
                 W2p, b2p.reshape(1, -1), W2n, b2n.reshape(1, -1), h2)

    s3 = _make_mid(128, 80, 2)(z2, src80, dst80, zc(128))
    z3a, z3b = _tc_mid(64, 128, True, s3, s1, z2,
                       W3p, b3p.reshape(1, -1), W3n, b3n.reshape(1, -1), h3)

    s4 = _make_l4()(z3a, z3b, src80, dst80, zc(128))
    return _tc_l4q(s4, s1, z3a, z3b,
                   W4p, b4p.reshape(1, -1), W4n, b4n.reshape(1, -1), h4, mu)


# speedup vs baseline: 1.6420x; 1.0289x over previous
import functools

import jax
import jax.numpy as jnp
from jax import lax
from jax.experimental import pallas as pl
from jax.experimental.pallas import tpu as pltpu
from jax.experimental.pallas import tpu_sc as plsc

N = 10000
E = 320000
EP = E + N
IN_DIMS = [3, 32, 64, 128, 256]
N_CLUSTERS = 30

NC, NS = 2, 16
NW = NC * NS
EPT = 10240
EPAD = EPT * NW
NPAD = 10240
ROWS_PT = NPAD // NS

_mesh = plsc.VectorSubcoreMesh(
    core_axis_name="c", subcore_axis_name="s", num_cores=NC, num_subcores=NS)
_sc_params = pltpu.CompilerParams(use_tc_tiling_on_sc=False)


def _geom(cc, kk):
    nch_arr = EPT // cc

    def bound(real):
        gch = -(-real // cc)
        b = -(-gch // NW)
        return min(-(-b // kk) * kk, nch_arr)

    return nch_arr, bound(E)


def _stage_idx(src_h, dst_h, core, sub, idxs, idxd):
    pltpu.sync_copy(src_h.at[core, sub], idxs)
    pltpu.sync_copy(dst_h.at[core, sub], idxd)


def _zero_acc(zc, acc, sub):
    pltpu.sync_copy(zc, acc.at[pl.ds(sub * ROWS_PT, ROWS_PT)])


def _copy_out(acc, out_ref, sub):
    pltpu.sync_copy(acc.at[pl.ds(sub * ROWS_PT, ROWS_PT)],
                    out_ref.at[pl.ds(sub * ROWS_PT, ROWS_PT)])


def _pipe_phase(tbl, idxs, idxd, bufs, sems, acc, nch, kk):
    @pl.loop(0, nch, step=kk)
    def _(j):
        descs = [
            pltpu.async_copy(tbl.at[idxs.at[j + b]], bufs[b], sems[b])
            for b in range(kk)
        ]
        for b in range(kk):
            descs[b].wait()
            pltpu.sync_copy(bufs[b], acc.at[idxd.at[j + b]], add=True)


def _make_mid(w, cc, kk):
    nch_arr, nch_e = _geom(cc, kk)

    @functools.partial(
        pl.kernel,
        out_type=jax.ShapeDtypeStruct((NC, NPAD, w), jnp.float32),
        mesh=_mesh,
        compiler_params=_sc_params,
        scratch_types=[
            pltpu.VMEM((nch_arr, cc), jnp.int32),
            pltpu.VMEM((nch_arr, cc), jnp.int32),
            [pltpu.VMEM((cc, w), jnp.float32)] * kk,
            pltpu.VMEM_SHARED((NPAD, w), jnp.float32),
            [pltpu.SemaphoreType.DMA] * kk,
        ],
    )
    def k(tbl, src_h, dst_h, zc, out, idxs, idxd, bufs, acc, sems):
        core = lax.axis_index("c")
        sub = lax.axis_index("s")
        nch = nch_e
        _stage_idx(src_h, dst_h, core, sub, idxs, idxd)
        _zero_acc(zc, acc, sub)
        plsc.subcore_barrier()
        _pipe_phase(tbl, idxs, idxd, bufs, sems, acc, nch, kk)
        plsc.subcore_barrier()
        _copy_out(acc, out.at[core], sub)
        plsc.subcore_barrier()

    return k


def _make_l4(cc=64, kk=3):
    d = 128
    nch_arr, nch_e = _geom(cc, kk)

    @functools.partial(
        pl.kernel,
        out_type=jax.ShapeDtypeStruct((NC, 2, NPAD, d), jnp.float32),
        mesh=_mesh,
        compiler_params=_sc_params,
        scratch_types=[
            pltpu.VMEM((nch_arr, cc), jnp.int32),
            pltpu.VMEM((nch_arr, cc), jnp.int32),
            [pltpu.VMEM((cc, d), jnp.float32)] * kk,
            pltpu.VMEM_SHARED((NPAD, d), jnp.float32),
            [pltpu.SemaphoreType.DMA] * kk,
        ],
    )
    def k(t1, t2, src_h, dst_h, zc, out, idxs, idxd, bufs, acc, sems):
        core = lax.axis_index("c")
        sub = lax.axis_index("s")
        nch = nch_e
        _stage_idx(src_h, dst_h, core, sub, idxs, idxd)
        for p, tp in ((0, t1), (1, t2)):
            _zero_acc(zc, acc, sub)
            plsc.subcore_barrier()
            _pipe_phase(tp, idxs, idxd, bufs, sems, acc, nch, kk)
            plsc.subcore_barrier()
            _copy_out(acc, out.at[core, p], sub)
        plsc.subcore_barrier()

    return k


_TCB = 2000


def _dot(a, b):
    return jnp.dot(a, b, preferred_element_type=jnp.float32)


def _l1_body(so, x, wp, bp, wn, bn, h, o):
    sov = so[...]
    rcp = 1.0 / (sov[0, :, 3:4] + 1.0)
    rcn = 1.0 / jnp.maximum(sov[1, :, 3:4], 1.0)
    xv = x[...]
    agg_p = (sov[0, :, :3] + xv) * rcp
    agg_n = sov[1, :, :3] * rcn
    wpv = wp[...]
    wnv = wn[...]
    out_p = _dot(agg_p, wpv[:3]) + _dot(xv, wpv[3:]) + bp[...]
    out_n = _dot(agg_n, wnv[:3]) + _dot(xv, wnv[3:]) + bn[...]
    hv = h[...]
    o[...] = jnp.concatenate(
        [(jnp.maximum(out_p, 0.0) + hv) * 0.5,
         (jnp.maximum(out_n, 0.0) + hv) * 0.5], axis=1)


def _mid_body(d, split, sp, sn, co, z, wp, bp, wn, bn, h, *outs):
    rcp = 1.0 / (co[...][0, :, 3:4] + 1.0)
    rcn = 1.0 / jnp.maximum(co[...][1, :, 3:4], 1.0)
    spv = sp[...]
    snv = sn[...]
    zv = z[...]
    ap1 = (spv[:, :d] + zv[:, :d]) * rcp
    ap2 = (spv[:, d:] + zv[:, d:]) * rcp
    an1 = snv[:, :d] * rcn
    an2 = snv[:, d:] * rcn
    wpv = wp[...]
    wnv = wn[...]
    out_p = (_dot(ap1, wpv[:d]) + _dot(an2, wpv[d:2 * d])
             + _dot(zv[:, :d], wpv[2 * d:]) + bp[...])
    out_n = (_dot(ap2, wnv[:d]) + _dot(an1, wnv[d:2 * d])
             + _dot(zv[:, d:], wnv[2 * d:]) + bn[...])
    hv = h[...]
    zp = (jnp.maximum(out_p, 0.0) + hv) * 0.5
    zn = (jnp.maximum(out_n, 0.0) + hv) * 0.5
    if split:
        outs[0][...] = zp
        outs[1][...] = zn
    else:
        outs[0][...] = jnp.concatenate([zp, zn], axis=1)


def _l4q_body(d, s4, co, z1, z2, wp, bp, wn, bn, h, mu, q1, q2):
    rcp = 1.0 / (co[...][0, :, 3:4] + 1.0)
    rcn = 1.0 / jnp.maximum(co[...][1, :, 3:4], 1.0)
    s4v = s4[...]
    ap1 = (s4v[0, 0] + z1[...]) * rcp
    ap2 = (s4v[0, 1] + z2[...]) * rcp
    an1 = s4v[1, 0] * rcn
    an2 = s4v[1, 1] * rcn
    wpv = wp[...]
    wnv = wn[...]
    out_p = (_dot(ap1, wpv[:d]) + _dot(an2, wpv[d:2 * d])
             + _dot(z1[...], wpv[2 * d:]) + bp[...])
    out_n = (_dot(ap2, wnv[:d]) + _dot(an1, wnv[d:2 * d])
             + _dot(z2[...], wnv[2 * d:]) + bn[...])
    hv = h[...]
    zp = (jnp.maximum(out_p, 0.0) + hv) * 0.5
    zn = (jnp.maximum(out_n, 0.0) + hv) * 0.5
    muv = mu[...]
    mu2 = jnp.sum(muv * muv, axis=1)[None, :]
    for zv, q in ((zp, q1), (zn, q2)):
        z2s = jnp.sum(zv * zv, axis=1, keepdims=True)
        cross = lax.dot_general(zv, muv, (((1,), (1,)), ((), ())),
                                preferred_element_type=jnp.float32)
        d2 = z2s + mu2 - 2.0 * cross
        qv = 1.0 / (1.0 + jnp.maximum(d2, 0.0))
        q[...] = qv / jnp.sum(qv, axis=1, keepdims=True)


def _row_spec(cols):
    return pl.BlockSpec((_TCB, cols), lambda i: (i, 0))


def _full_spec(r, c):
    return pl.BlockSpec((r, c), lambda i: (0, 0))


_s1_spec = pl.BlockSpec((2, _TCB, 16), lambda i: (0, i, 0))


def _tc_layer1(s1, x, wp, bp, wn, bn, h):
    g = N // _TCB
    dout = 32
    return pl.pallas_call(
        _l1_body,
        grid=(g,),
        in_specs=[
            _s1_spec,
            _row_spec(3),
            _full_spec(6, dout), _full_spec(1, dout),
            _full_spec(6, dout), _full_spec(1, dout),
            _row_spec(dout),
        ],
        out_specs=pl.BlockSpec((_TCB, 2 * dout), lambda i: (i, 0)),
        out_shape=jax.ShapeDtypeStruct((N, 2 * dout), jnp.float32),
    )(s1, x, wp, bp, wn, bn, h)


def _tc_mid(d, dout, split, s, s1, z, wp, bp, wn, bn, h):
    g = N // _TCB
    if split:
        out_specs = [_row_spec(dout), _row_spec(dout)]
        out_shape = [jax.ShapeDtypeStruct((N, dout), jnp.float32)] * 2
    else:
        out_specs = pl.BlockSpec((_TCB, 2 * dout), lambda i: (i, 0))
        out_shape = jax.ShapeDtypeStruct((N, 2 * dout), jnp.float32)
    return pl.pallas_call(
        functools.partial(_mid_body, d, split),
        grid=(g,),
        in_specs=[
            pl.BlockSpec((None, _TCB, 2 * d), lambda i: (0, i, 0)),
            pl.BlockSpec((None, _TCB, 2 * d), lambda i: (1, i, 0)),
            _s1_spec,
            _row_spec(2 * d),
            _full_spec(3 * d, dout), _full_spec(1, dout),
            _full_spec(3 * d, dout), _full_spec(1, dout),
            _row_spec(dout),
        ],
        out_specs=out_specs,
        out_shape=out_shape,
    )(s, s, s1, z, wp, bp, wn, bn, h)


def _tc_l4q(s4, s1, z1, z2, wp, bp, wn, bn, h, mu):
    g = N // _TCB
    d, dout = 128, 256
    return pl.pallas_call(
        functools.partial(_l4q_body, d),
        grid=(g,),
        in_specs=[
            pl.BlockSpec((2, 2, _TCB, d), lambda i: (0, 0, i, 0)),
            _s1_spec,
            _row_spec(d), _row_spec(d),
            _full_spec(3 * d, dout), _full_spec(1, dout),
            _full_spec(3 * d, dout), _full_spec(1, dout),
            _row_spec(dout),
            _full_spec(N_CLUSTERS, dout),
        ],
        out_specs=[_row_spec(N_CLUSTERS), _row_spec(N_CLUSTERS)],
        out_shape=[jax.ShapeDtypeStruct((N, N_CLUSTERS), jnp.float32)] * 2,
    )(s4, s1, z1, z2, wp, bp, wn, bn, h, mu)


def _pad_edges(src, dst):
    pad = EPAD - src.shape[0]
    src_p = jnp.concatenate([src, jnp.zeros((pad,), jnp.int32)])
    dst_p = jnp.concatenate([dst, jnp.full((pad,), N, jnp.int32)])
    return src_p, dst_p


def _tile_layout(arr, cc):
    return arr.reshape(NW, EPT // cc, cc)


def kernel(x, edge_index_sim, edge_index_disim, h1, h2, h3, h4,
           W1p, b1p, W1n, b1n, W2p, b2p, W2n, b2n, W3p, b3p, W3n, b3n,
           W4p, b4p, W4n, b4n, mu):
    sp_s, dp_s = _pad_edges(edge_index_sim[0], edge_index_sim[1])
    sn_s, dn_s = _pad_edges(edge_index_disim[0], edge_index_disim[1])
    src128 = jnp.stack([_tile_layout(sp_s, 128), _tile_layout(sn_s, 128)])
    dst128 = jnp.stack([_tile_layout(dp_s, 128), _tile_layout(dn_s, 128)])
    src64 = jnp.stack([_tile_layout(sp_s, 64), _tile_layout(sn_s, 64)])
    dst64 = jnp.stack([_tile_layout(dp_s, 64), _tile_layout(dn_s, 64)])

    x16 = jnp.pad(x, ((0, 0), (0, 13))).at[:, 3].set(1.0)

    def zc(w):
        return jnp.zeros((ROWS_PT, w), jnp.float32)

    s1 = _make_mid(16, 128, 4)(x16, src128, dst128, zc(16))
    z1 = _tc_layer1(s1, x, W1p, b1p.reshape(1, -1), W1n, b1n.reshape(1, -1),
                    h1)

    s2 = _make_mid(64, 128, 4)(z1, src128, dst128, zc(64))
    z2 = _tc_mid(32, 64, False, s2, s1, z1,
                 W2p, b2p.reshape(1, -1), W2n, b2n.reshape(1, -1), h2)

    s3 = _make_mid(128, 64, 3)(z2, src64, dst64, zc(128))
    z3a, z3b = _tc_mid(64, 128, True, s3, s1, z2,
                       W3p, b3p.reshape(1, -1), W3n, b3n.reshape(1, -1), h3)

    s4 = _make_l4()(z3a, z3b, src64, dst64, zc(128))
    return _tc_l4q(s4, s1, z3a, z3b,
                   W4p, b4p.reshape(1, -1), W4n, b4n.reshape(1, -1), h4, mu)


# speedup vs baseline: 1.6758x; 1.0206x over previous
import functools

import jax
import jax.numpy as jnp
from jax import lax
from jax.experimental import pallas as pl
from jax.experimental.pallas import tpu as pltpu
from jax.experimental.pallas import tpu_sc as plsc

N = 10000
E = 320000
EP = E + N
IN_DIMS = [3, 32, 64, 128, 256]
N_CLUSTERS = 30

NC, NS = 2, 16
NW = NC * NS
EPT = 10240
EPAD = EPT * NW
NPAD = 10240
ROWS_PT = NPAD // NS

_mesh = plsc.VectorSubcoreMesh(
    core_axis_name="c", subcore_axis_name="s", num_cores=NC, num_subcores=NS)
_sc_params = pltpu.CompilerParams(use_tc_tiling_on_sc=False)


def _geom(cc, kk):
    nch_arr = EPT // cc

    def bound(real):
        gch = -(-real // cc)
        b = -(-gch // NW)
        return min(-(-b // kk) * kk, nch_arr)

    return nch_arr, bound(E)


def _stage_idx(src_h, dst_h, core, sub, idxs, idxd):
    pltpu.sync_copy(src_h.at[core, sub], idxs)
    pltpu.sync_copy(dst_h.at[core, sub], idxd)


def _zero_acc(zc, acc, sub):
    pltpu.sync_copy(zc, acc.at[pl.ds(sub * ROWS_PT, ROWS_PT)])


def _copy_out(acc, out_ref, sub):
    pltpu.sync_copy(acc.at[pl.ds(sub * ROWS_PT, ROWS_PT)],
                    out_ref.at[pl.ds(sub * ROWS_PT, ROWS_PT)])


def _pipe_phase(tbl, idxs, idxd, bufs, sems, acc, nch, kk):
    @pl.loop(0, nch, step=kk)
    def _(j):
        descs = [
            pltpu.async_copy(tbl.at[idxs.at[j + b]], bufs[b], sems[b])
            for b in range(kk)
        ]
        for b in range(kk):
            descs[b].wait()
            pltpu.sync_copy(bufs[b], acc.at[idxd.at[j + b]], add=True)


def _make_mid(w, cc, kk):
    nch_arr, nch_e = _geom(cc, kk)

    @functools.partial(
        pl.kernel,
        out_type=jax.ShapeDtypeStruct((NC, NPAD, w), jnp.float32),
        mesh=_mesh,
        compiler_params=_sc_params,
        scratch_types=[
            pltpu.VMEM((nch_arr, cc), jnp.int32),
            pltpu.VMEM((nch_arr, cc), jnp.int32),
            [pltpu.VMEM((cc, w), jnp.float32)] * kk,
            pltpu.VMEM_SHARED((NPAD, w), jnp.float32),
            [pltpu.SemaphoreType.DMA] * kk,
        ],
    )
    def k(tbl, src_h, dst_h, zc, out, idxs, idxd, bufs, acc, sems):
        core = lax.axis_index("c")
        sub = lax.axis_index("s")
        nch = nch_e
        _stage_idx(src_h, dst_h, core, sub, idxs, idxd)
        _zero_acc(zc, acc, sub)
        plsc.subcore_barrier()
        _pipe_phase(tbl, idxs, idxd, bufs, sems, acc, nch, kk)
        plsc.subcore_barrier()
        _copy_out(acc, out.at[core], sub)
        plsc.subcore_barrier()

    return k


def _make_l4(cc=64, kk=3):
    d = 128
    nch_arr, nch_e = _geom(cc, kk)

    @functools.partial(
        pl.kernel,
        out_type=jax.ShapeDtypeStruct((NC, 2, NPAD, d), jnp.float32),
        mesh=_mesh,
        compiler_params=_sc_params,
        scratch_types=[
            pltpu.VMEM((nch_arr, cc), jnp.int32),
            pltpu.VMEM((nch_arr, cc), jnp.int32),
            [pltpu.VMEM((cc, d), jnp.float32)] * kk,
            pltpu.VMEM_SHARED((NPAD, d), jnp.float32),
            [pltpu.SemaphoreType.DMA] * kk,
        ],
    )
    def k(t1, t2, src_h, dst_h, zc, out, idxs, idxd, bufs, acc, sems):
        core = lax.axis_index("c")
        sub = lax.axis_index("s")
        nch = nch_e
        _stage_idx(src_h, dst_h, core, sub, idxs, idxd)
        for p, tp in ((0, t1), (1, t2)):
            _zero_acc(zc, acc, sub)
            plsc.subcore_barrier()
            _pipe_phase(tp, idxs, idxd, bufs, sems, acc, nch, kk)
            plsc.subcore_barrier()
            _copy_out(acc, out.at[core, p], sub)
        plsc.subcore_barrier()

    return k


_TCB = 2000


def _dot(a, b):
    return jnp.dot(a, b, preferred_element_type=jnp.float32)


def _l1_body(so, x, wp, bp, wn, bn, h, o):
    sov = so[...]
    rcp = 1.0 / (sov[0, :, 3:4] + 1.0)
    rcn = 1.0 / jnp.maximum(sov[1, :, 3:4], 1.0)
    xv = x[...]
    agg_p = (sov[0, :, :3] + xv) * rcp
    agg_n = sov[1, :, :3] * rcn
    wpv = wp[...]
    wnv = wn[...]
    out_p = _dot(agg_p, wpv[:3]) + _dot(xv, wpv[3:]) + bp[...]
    out_n = _dot(agg_n, wnv[:3]) + _dot(xv, wnv[3:]) + bn[...]
    hv = h[...]
    o[...] = jnp.concatenate(
        [(jnp.maximum(out_p, 0.0) + hv) * 0.5,
         (jnp.maximum(out_n, 0.0) + hv) * 0.5], axis=1)


def _mid_body(d, split, sp, sn, co, z, wp, bp, wn, bn, h, *outs):
    rcp = 1.0 / (co[...][0, :, 3:4] + 1.0)
    rcn = 1.0 / jnp.maximum(co[...][1, :, 3:4], 1.0)
    spv = sp[...]
    snv = sn[...]
    zv = z[...]
    ap1 = (spv[:, :d] + zv[:, :d]) * rcp
    ap2 = (spv[:, d:] + zv[:, d:]) * rcp
    an1 = snv[:, :d] * rcn
    an2 = snv[:, d:] * rcn
    wpv = wp[...]
    wnv = wn[...]
    out_p = (_dot(ap1, wpv[:d]) + _dot(an2, wpv[d:2 * d])
             + _dot(zv[:, :d], wpv[2 * d:]) + bp[...])
    out_n = (_dot(ap2, wnv[:d]) + _dot(an1, wnv[d:2 * d])
             + _dot(zv[:, d:], wnv[2 * d:]) + bn[...])
    hv = h[...]
    zp = (jnp.maximum(out_p, 0.0) + hv) * 0.5
    zn = (jnp.maximum(out_n, 0.0) + hv) * 0.5
    if split:
        outs[0][...] = zp
        outs[1][...] = zn
    else:
        outs[0][...] = jnp.concatenate([zp, zn], axis=1)


def _l4q_body(d, s4, co, z1, z2, wp, bp, wn, bn, h, mu, q1, q2):
    rcp = 1.0 / (co[...][0, :, 3:4] + 1.0)
    rcn = 1.0 / jnp.maximum(co[...][1, :, 3:4], 1.0)
    s4v = s4[...]
    ap1 = (s4v[0, 0] + z1[...]) * rcp
    ap2 = (s4v[0, 1] + z2[...]) * rcp
    an1 = s4v[1, 0] * rcn
    an2 = s4v[1, 1] * rcn
    wpv = wp[...]
    wnv = wn[...]
    out_p = (_dot(ap1, wpv[:d]) + _dot(an2, wpv[d:2 * d])
             + _dot(z1[...], wpv[2 * d:]) + bp[...])
    out_n = (_dot(ap2, wnv[:d]) + _dot(an1, wnv[d:2 * d])
             + _dot(z2[...], wnv[2 * d:]) + bn[...])
    hv = h[...]
    zp = (jnp.maximum(out_p, 0.0) + hv) * 0.5
    zn = (jnp.maximum(out_n, 0.0) + hv) * 0.5
    muv = mu[...]
    mu2 = jnp.sum(muv * muv, axis=1)[None, :]
    for zv, q in ((zp, q1), (zn, q2)):
        z2s = jnp.sum(zv * zv, axis=1, keepdims=True)
        cross = lax.dot_general(zv, muv, (((1,), (1,)), ((), ())),
                                preferred_element_type=jnp.float32)
        d2 = z2s + mu2 - 2.0 * cross
        qv = 1.0 / (1.0 + jnp.maximum(d2, 0.0))
        q[...] = qv / jnp.sum(qv, axis=1, keepdims=True)


def _row_spec(cols):
    return pl.BlockSpec((_TCB, cols), lambda i: (i, 0))


def _full_spec(r, c):
    return pl.BlockSpec((r, c), lambda i: (0, 0))


_s1_spec = pl.BlockSpec((2, _TCB, 16), lambda i: (0, i, 0))


def _tc_layer1(s1, x, wp, bp, wn, bn, h):
    g = N // _TCB
    dout = 32
    return pl.pallas_call(
        _l1_body,
        grid=(g,),
        in_specs=[
            _s1_spec,
            _row_spec(3),
            _full_spec(6, dout), _full_spec(1, dout),
            _full_spec(6, dout), _full_spec(1, dout),
            _row_spec(dout),
        ],
        out_specs=pl.BlockSpec((_TCB, 2 * dout), lambda i: (i, 0)),
        out_shape=jax.ShapeDtypeStruct((N, 2 * dout), jnp.float32),
    )(s1, x, wp, bp, wn, bn, h)


def _tc_mid(d, dout, split, s, s1, z, wp, bp, wn, bn, h):
    g = N // _TCB
    if split:
        out_specs = [_row_spec(dout), _row_spec(dout)]
        out_shape = [jax.ShapeDtypeStruct((N, dout), jnp.float32)] * 2
    else:
        out_specs = pl.BlockSpec((_TCB, 2 * dout), lambda i: (i, 0))
        out_shape = jax.ShapeDtypeStruct((N, 2 * dout), jnp.float32)
    return pl.pallas_call(
        functools.partial(_mid_body, d, split),
        grid=(g,),
        in_specs=[
            pl.BlockSpec((None, _TCB, 2 * d), lambda i: (0, i, 0)),
            pl.BlockSpec((None, _TCB, 2 * d), lambda i: (1, i, 0)),
            _s1_spec,
            _row_spec(2 * d),
            _full_spec(3 * d, dout), _full_spec(1, dout),
            _full_spec(3 * d, dout), _full_spec(1, dout),
            _row_spec(dout),
        ],
        out_specs=out_specs,
        out_shape=out_shape,
    )(s, s, s1, z, wp, bp, wn, bn, h)


def _tc_l4q(s4, s1, z1, z2, wp, bp, wn, bn, h, mu):
    g = N // _TCB
    d, dout = 128, 256
    return pl.pallas_call(
        functools.partial(_l4q_body, d),
        grid=(g,),
        in_specs=[
            pl.BlockSpec((2, 2, _TCB, d), lambda i: (0, 0, i, 0)),
            _s1_spec,
            _row_spec(d), _row_spec(d),
            _full_spec(3 * d, dout), _full_spec(1, dout),
            _full_spec(3 * d, dout), _full_spec(1, dout),
            _row_spec(dout),
            _full_spec(N_CLUSTERS, dout),
        ],
        out_specs=[_row_spec(N_CLUSTERS), _row_spec(N_CLUSTERS)],
        out_shape=[jax.ShapeDtypeStruct((N, N_CLUSTERS), jnp.float32)] * 2,
    )(s4, s1, z1, z2, wp, bp, wn, bn, h, mu)


def _pad_edges(src, dst):
    pad = EPAD - src.shape[0]
    src_p = jnp.concatenate([src, jnp.zeros((pad,), jnp.int32)])
    dst_p = jnp.concatenate([dst, jnp.full((pad,), N, jnp.int32)])
    return src_p, dst_p


def _tile_layout(arr, cc):
    return arr.reshape(NW, EPT // cc, cc)


def kernel(x, edge_index_sim, edge_index_disim, h1, h2, h3, h4,
           W1p, b1p, W1n, b1n, W2p, b2p, W2n, b2n, W3p, b3p, W3n, b3n,
           W4p, b4p, W4n, b4n, mu):
    sp_s, dp_s = _pad_edges(edge_index_sim[0], edge_index_sim[1])
    sn_s, dn_s = _pad_edges(edge_index_disim[0], edge_index_disim[1])
    src128 = jnp.stack([_tile_layout(sp_s, 128), _tile_layout(sn_s, 128)])
    dst128 = jnp.stack([_tile_layout(dp_s, 128), _tile_layout(dn_s, 128)])
    src64 = jnp.stack([_tile_layout(sp_s, 64), _tile_layout(sn_s, 64)])
    dst64 = jnp.stack([_tile_layout(dp_s, 64), _tile_layout(dn_s, 64)])

    x16 = jnp.pad(x, ((0, 0), (0, 13))).at[:, 3].set(1.0)

    def zc(w):
        return jnp.zeros((ROWS_PT, w), jnp.float32)

    s1 = _make_mid(16, 128, 8)(x16, src128, dst128, zc(16))
    z1 = _tc_layer1(s1, x, W1p, b1p.reshape(1, -1), W1n, b1n.reshape(1, -1),
                    h1)

    s2 = _make_mid(64, 128, 8)(z1, src128, dst128, zc(64))
    z2 = _tc_mid(32, 64, False, s2, s1, z1,
                 W2p, b2p.reshape(1, -1), W2n, b2n.reshape(1, -1), h2)

    s3 = _make_mid(128, 64, 3)(z2, src64, dst64, zc(128))
    z3a, z3b = _tc_mid(64, 128, True, s3, s1, z2,
                       W3p, b3p.reshape(1, -1), W3n, b3n.reshape(1, -1), h3)

    s4 = _make_l4()(z3a, z3b, src64, dst64, zc(128))
    return _tc_l4q(s4, s1, z3a, z3b,
                   W4p, b4p.reshape(1, -1), W4n, b4n.reshape(1, -1), h4, mu)


# speedup vs baseline: 1.6768x; 1.0006x over previous
import functools

import jax
import jax.numpy as jnp
from jax import lax
from jax.experimental import pallas as pl
from jax.experimental.pallas import tpu as pltpu
from jax.experimental.pallas import tpu_sc as plsc

N = 10000
E = 320000
IN_DIMS = [3, 32, 64, 128, 256]
N_CLUSTERS = 30

NC, NS = 2, 16
NW = NC * NS
EPT = 10240
EPAD = EPT * NW
NPAD = 10240
ROWS_PT = NPAD // NS

_mesh = plsc.VectorSubcoreMesh(
    core_axis_name="c", subcore_axis_name="s", num_cores=NC, num_subcores=NS)
_sc_params = pltpu.CompilerParams(use_tc_tiling_on_sc=False)


def _geom(cc, kk):
    nch_arr = EPT // cc

    def bound(real):
        gch = -(-real // cc)
        b = -(-gch // NW)
        return min(-(-b // kk) * kk, nch_arr)

    return nch_arr, bound(E)


def _stage_idx(src_h, dst_h, core, sub, idxs, idxd):
    pltpu.sync_copy(src_h.at[core, sub], idxs)
    pltpu.sync_copy(dst_h.at[core, sub], idxd)


def _zero_acc(zc, acc, sub):
    pltpu.sync_copy(zc, acc.at[pl.ds(sub * ROWS_PT, ROWS_PT)])


def _copy_out(acc, out_ref, sub):
    pltpu.sync_copy(acc.at[pl.ds(sub * ROWS_PT, ROWS_PT)],
                    out_ref.at[pl.ds(sub * ROWS_PT, ROWS_PT)])


def _pipe_phase(tbl, idxs, idxd, bufs, sems, acc, nch, kk):
    @pl.loop(0, nch, step=kk)
    def _(j):
        descs = [
            pltpu.async_copy(tbl.at[idxs.at[j + b]], bufs[b], sems[b])
            for b in range(kk)
        ]
        for b in range(kk):
            descs[b].wait()
            pltpu.sync_copy(bufs[b], acc.at[idxd.at[j + b]], add=True)


def _make_mid(w, cc, kk):
    nch_arr, nch_e = _geom(cc, kk)

    @functools.partial(
        pl.kernel,
        out_type=jax.ShapeDtypeStruct((NC, NPAD, w), jnp.float32),
        mesh=_mesh,
        compiler_params=_sc_params,
        scratch_types=[
            pltpu.VMEM((nch_arr, cc), jnp.int32),
            pltpu.VMEM((nch_arr, cc), jnp.int32),
            [pltpu.VMEM((cc, w), jnp.float32)] * kk,
            pltpu.VMEM_SHARED((NPAD, w), jnp.float32),
            [pltpu.SemaphoreType.DMA] * kk,
        ],
    )
    def k(tbl, src_h, dst_h, zc, out, idxs, idxd, bufs, acc, sems):
        core = lax.axis_index("c")
        sub = lax.axis_index("s")
        nch = nch_e
        _stage_idx(src_h, dst_h, core, sub, idxs, idxd)
        _zero_acc(zc, acc, sub)
        plsc.subcore_barrier()
        _pipe_phase(tbl, idxs, idxd, bufs, sems, acc, nch, kk)
        plsc.subcore_barrier()
        _copy_out(acc, out.at[core], sub)
        plsc.subcore_barrier()

    return k


def _make_l4(cc=64, kk=3):
    d = 128
    nch_arr, nch_e = _geom(cc, kk)

    @functools.partial(
        pl.kernel,
        out_type=jax.ShapeDtypeStruct((NC, 2, NPAD, d), jnp.float32),
        mesh=_mesh,
        compiler_params=_sc_params,
        scratch_types=[
            pltpu.VMEM((nch_arr, cc), jnp.int32),
            pltpu.VMEM((nch_arr, cc), jnp.int32),
            [pltpu.VMEM((cc, d), jnp.float32)] * kk,
            pltpu.VMEM_SHARED((NPAD, d), jnp.float32),
            [pltpu.SemaphoreType.DMA] * kk,
        ],
    )
    def k(t1, t2, src_h, dst_h, zc, out, idxs, idxd, bufs, acc, sems):
        core = lax.axis_index("c")
        sub = lax.axis_index("s")
        nch = nch_e
        _stage_idx(src_h, dst_h, core, sub, idxs, idxd)
        for p, tp in ((0, t1), (1, t2)):
            _zero_acc(zc, acc, sub)
            plsc.subcore_barrier()
            _pipe_phase(tp, idxs, idxd, bufs, sems, acc, nch, kk)
            plsc.subcore_barrier()
            _copy_out(acc, out.at[core, p], sub)
        plsc.subcore_barrier()

    return k


_TCB = 2000


def _dot(a, b):
    return jnp.dot(a, b, preferred_element_type=jnp.float32)


def _l1_body(so, x, wp, bp, wn, bn, h, o):
    sov = so[...]
    rcp = 1.0 / (sov[0, :, 3:4] + 1.0)
    rcn = 1.0 / jnp.maximum(sov[1, :, 3:4], 1.0)
    xv = x[...]
    agg_p = (sov[0, :, :3] + xv) * rcp
    agg_n = sov[1, :, :3] * rcn
    wpv = wp[...]
    wnv = wn[...]
    out_p = _dot(agg_p, wpv[:3]) + _dot(xv, wpv[3:]) + bp[...]
    out_n = _dot(agg_n, wnv[:3]) + _dot(xv, wnv[3:]) + bn[...]
    hv = h[...]
    o[...] = jnp.concatenate(
        [(jnp.maximum(out_p, 0.0) + hv) * 0.5,
         (jnp.maximum(out_n, 0.0) + hv) * 0.5], axis=1)


def _mid_body(d, split, sp, sn, co, z, wp, bp, wn, bn, h, *outs):
    rcp = 1.0 / (co[...][0, :, 3:4] + 1.0)
    rcn = 1.0 / jnp.maximum(co[...][1, :, 3:4], 1.0)
    spv = sp[...]
    snv = sn[...]
    zv = z[...]
    ap1 = (spv[:, :d] + zv[:, :d]) * rcp
    ap2 = (spv[:, d:] + zv[:, d:]) * rcp
    an1 = snv[:, :d] * rcn
    an2 = snv[:, d:] * rcn
    wpv = wp[...]
    wnv = wn[...]
    out_p = (_dot(ap1, wpv[:d]) + _dot(an2, wpv[d:2 * d])
             + _dot(zv[:, :d], wpv[2 * d:]) + bp[...])
    out_n = (_dot(ap2, wnv[:d]) + _dot(an1, wnv[d:2 * d])
             + _dot(zv[:, d:], wnv[2 * d:]) + bn[...])
    hv = h[...]
    zp = (jnp.maximum(out_p, 0.0) + hv) * 0.5
    zn = (jnp.maximum(out_n, 0.0) + hv) * 0.5
    if split:
        outs[0][...] = zp
        outs[1][...] = zn
    else:
        outs[0][...] = jnp.concatenate([zp, zn], axis=1)


def _l4q_body(d, s4, co, z1, z2, wp, bp, wn, bn, h, mu, q1, q2):
    rcp = 1.0 / (co[...][0, :, 3:4] + 1.0)
    rcn = 1.0 / jnp.maximum(co[...][1, :, 3:4], 1.0)
    s4v = s4[...]
    ap1 = (s4v[0, 0] + z1[...]) * rcp
    ap2 = (s4v[0, 1] + z2[...]) * rcp
    an1 = s4v[1, 0] * rcn
    an2 = s4v[1, 1] * rcn
    wpv = wp[...]
    wnv = wn[...]
    out_p = (_dot(ap1, wpv[:d]) + _dot(an2, wpv[d:2 * d])
             + _dot(z1[...], wpv[2 * d:]) + bp[...])
    out_n = (_dot(ap2, wnv[:d]) + _dot(an1, wnv[d:2 * d])
             + _dot(z2[...], wnv[2 * d:]) + bn[...])
    hv = h[...]
    zp = (jnp.maximum(out_p, 0.0) + hv) * 0.5
    zn = (jnp.maximum(out_n, 0.0) + hv) * 0.5
    muv = mu[...]
    mu2 = jnp.sum(muv * muv, axis=1)[None, :]
    for zv, q in ((zp, q1), (zn, q2)):
        z2s = jnp.sum(zv * zv, axis=1, keepdims=True)
        cross = lax.dot_general(zv, muv, (((1,), (1,)), ((), ())),
                                preferred_element_type=jnp.float32)
        d2 = z2s + mu2 - 2.0 * cross
        qv = 1.0 / (1.0 + jnp.maximum(d2, 0.0))
        q[...] = qv / jnp.sum(qv, axis=1, keepdims=True)


def _row_spec(cols):
    return pl.BlockSpec((_TCB, cols), lambda i: (i, 0))


def _full_spec(r, c):
    return pl.BlockSpec((r, c), lambda i: (0, 0))


_s1_spec = pl.BlockSpec((2, _TCB, 16), lambda i: (0, i, 0))


def _tc_layer1(s1, x, wp, bp, wn, bn, h):
    g = N // _TCB
    dout = 32
    return pl.pallas_call(
        _l1_body,
        grid=(g,),
        in_specs=[
            _s1_spec,
            _row_spec(3),
            _full_spec(6, dout), _full_spec(1, dout),
            _full_spec(6, dout), _full_spec(1, dout),
            _row_spec(dout),
        ],
        out_specs=pl.BlockSpec((_TCB, 2 * dout), lambda i: (i, 0)),
        out_shape=jax.ShapeDtypeStruct((N, 2 * dout), jnp.float32),
    )(s1, x, wp, bp, wn, bn, h)


def _tc_mid(d, dout, split, s, s1, z, wp, bp, wn, bn, h):
    g = N // _TCB
    if split:
        out_specs = [_row_spec(dout), _row_spec(dout)]
        out_shape = [jax.ShapeDtypeStruct((N, dout), jnp.float32)] * 2
    else:
        out_specs = pl.BlockSpec((_TCB, 2 * dout), lambda i: (i, 0))
        out_shape = jax.ShapeDtypeStruct((N, 2 * dout), jnp.float32)
    return pl.pallas_call(
        functools.partial(_mid_body, d, split),
        grid=(g,),
        in_specs=[
            pl.BlockSpec((None, _TCB, 2 * d), lambda i: (0, i, 0)),
            pl.BlockSpec((None, _TCB, 2 * d), lambda i: (1, i, 0)),
            _s1_spec,
            _row_spec(2 * d),
            _full_spec(3 * d, dout), _full_spec(1, dout),
            _full_spec(3 * d, dout), _full_spec(1, dout),
            _row_spec(dout),
        ],
        out_specs=out_specs,
        out_shape=out_shape,
    )(s, s, s1, z, wp, bp, wn, bn, h)


def _tc_l4q(s4, s1, z1, z2, wp, bp, wn, bn, h, mu):
    g = N // _TCB
    d, dout = 128, 256
    return pl.pallas_call(
        functools.partial(_l4q_body, d),
        grid=(g,),
        in_specs=[
            pl.BlockSpec((2, 2, _TCB, d), lambda i: (0, 0, i, 0)),
            _s1_spec,
            _row_spec(d), _row_spec(d),
            _full_spec(3 * d, dout), _full_spec(1, dout),
            _full_spec(3 * d, dout), _full_spec(1, dout),
            _row_spec(dout),
            _full_spec(N_CLUSTERS, dout),
        ],
        out_specs=[_row_spec(N_CLUSTERS), _row_spec(N_CLUSTERS)],
        out_shape=[jax.ShapeDtypeStruct((N, N_CLUSTERS), jnp.float32)] * 2,
    )(s4, s1, z1, z2, wp, bp, wn, bn, h, mu)


def _pad_edges(src, dst):
    pad = EPAD - src.shape[0]
    src_p = jnp.concatenate([src, jnp.zeros((pad,), jnp.int32)])
    dst_p = jnp.concatenate([dst, jnp.full((pad,), N, jnp.int32)])
    return src_p, dst_p


def _tile_layout(arr, cc):
    return arr.reshape(NW, EPT // cc, cc)


def kernel(x, edge_index_sim, edge_index_disim, h1, h2, h3, h4,
           W1p, b1p, W1n, b1n, W2p, b2p, W2n, b2n, W3p, b3p, W3n, b3n,
           W4p, b4p, W4n, b4n, mu):
    sp_s, dp_s = _pad_edges(edge_index_sim[0], edge_index_sim[1])
    sn_s, dn_s = _pad_edges(edge_index_disim[0], edge_index_disim[1])
    src128 = jnp.stack([_tile_layout(sp_s, 128), _tile_layout(sn_s, 128)])
    dst128 = jnp.stack([_tile_layout(dp_s, 128), _tile_layout(dn_s, 128)])
    src64 = jnp.stack([_tile_layout(sp_s, 64), _tile_layout(sn_s, 64)])
    dst64 = jnp.stack([_tile_layout(dp_s, 64), _tile_layout(dn_s, 64)])

    x16 = jnp.pad(x, ((0, 0), (0, 13))).at[:, 3].set(1.0)

    def zc(w):
        return jnp.zeros((ROWS_PT, w), jnp.float32)

    s1 = _make_mid(16, 128, 8)(x16, src128, dst128, zc(16))
    z1 = _tc_layer1(s1, x, W1p, b1p.reshape(1, -1), W1n, b1n.reshape(1, -1),
                    h1)

    s2 = _make_mid(64, 128, 8)(z1, src128, dst128, zc(64))
    z2 = _tc_mid(32, 64, False, s2, s1, z1,
                 W2p, b2p.reshape(1, -1), W2n, b2n.reshape(1, -1), h2)

    s3 = _make_mid(128, 64, 3)(z2, src64, dst64, zc(128))
    z3a, z3b = _tc_mid(64, 128, True, s3, s1, z2,
                       W3p, b3p.reshape(1, -1), W3n, b3n.reshape(1, -1), h3)

    s4 = _make_l4()(z3a, z3b, src64, dst64, zc(128))
    return _tc_l4q(s4, s1, z3a, z3b,
                   W4p, b4p.reshape(1, -1), W4n, b4n.reshape(1, -1), h4, mu)
